# initial kernel scaffold (unmeasured)
import functools

import jax
import jax.numpy as jnp
from jax import lax
from jax.experimental import pallas as pl
from jax.experimental.pallas import tpu as pltpu

B, S, H, Dh, Dr = 4, 256, 32, 128, 64
D = 4096
DC_SH = 128
M = B * S
SCALE = (Dh + Dr) ** -0.5


def _kv_body(x_ref, wdkv_ref, wuk_ref, wuv_ref,
             k_ref, v_ref,
             c_mine, c_other, wuk_mine, wuk_other, wuv_mine, wuv_other,
             send_sems, recv_sems):
    my_x = lax.axis_index("x")
    my_y = lax.axis_index("y")
    nbr = (my_x, 1 - my_y)

    barrier_sem = pltpu.get_barrier_semaphore()
    pl.semaphore_signal(barrier_sem, inc=1, device_id=nbr,
                        device_id_type=pl.DeviceIdType.MESH)
    pl.semaphore_wait(barrier_sem, 1)

    wuk_mine[...] = wuk_ref[...].astype(jnp.bfloat16)
    wuv_mine[...] = wuv_ref[...].astype(jnp.bfloat16)
    rdma_wuk = pltpu.make_async_remote_copy(
        src_ref=wuk_mine, dst_ref=wuk_other,
        send_sem=send_sems.at[0], recv_sem=recv_sems.at[0],
        device_id=nbr, device_id_type=pl.DeviceIdType.MESH)
    rdma_wuk.start()
    rdma_wuv = pltpu.make_async_remote_copy(
        src_ref=wuv_mine, dst_ref=wuv_other,
        send_sem=send_sems.at[1], recv_sem=recv_sems.at[1],
        device_id=nbr, device_id_type=pl.DeviceIdType.MESH)
    rdma_wuv.start()

    c_mine[...] = jnp.dot(
        x_ref[...].astype(jnp.bfloat16), wdkv_ref[...].astype(jnp.bfloat16),
        preferred_element_type=jnp.float32).astype(jnp.bfloat16)
    rdma_c = pltpu.make_async_remote_copy(
        src_ref=c_mine, dst_ref=c_other,
        send_sem=send_sems.at[2], recv_sem=recv_sems.at[2],
        device_id=nbr, device_id_type=pl.DeviceIdType.MESH)
    rdma_c.start()

    rdma_wuk.wait()
    rdma_wuv.wait()
    rdma_c.wait()

    k_ref[...] = (
        jnp.dot(c_mine[...], wuk_mine[...], preferred_element_type=jnp.float32)
        + jnp.dot(c_other[...], wuk_other[...], preferred_element_type=jnp.float32)
    ).astype(jnp.bfloat16)
    v_ref[...] = (
        jnp.dot(c_mine[...], wuv_mine[...], preferred_element_type=jnp.float32)
        + jnp.dot(c_other[...], wuv_other[...], preferred_element_type=jnp.float32)
    ).astype(jnp.bfloat16)


def _kv_exchange(x2d, wdkv, wuk, wuv):
    return pl.pallas_call(
        _kv_body,
        out_shape=[
            jax.ShapeDtypeStruct((M, H * Dh), jnp.bfloat16),
            jax.ShapeDtypeStruct((M, H * Dh), jnp.bfloat16),
        ],
        in_specs=[pl.BlockSpec(memory_space=pltpu.VMEM)] * 4,
        out_specs=[pl.BlockSpec(memory_space=pltpu.VMEM)] * 2,
        scratch_shapes=[
            pltpu.VMEM((M, DC_SH), jnp.bfloat16),
            pltpu.VMEM((M, DC_SH), jnp.bfloat16),
            pltpu.VMEM((DC_SH, H * Dh), jnp.bfloat16),
            pltpu.VMEM((DC_SH, H * Dh), jnp.bfloat16),
            pltpu.VMEM((DC_SH, H * Dh), jnp.bfloat16),
            pltpu.VMEM((DC_SH, H * Dh), jnp.bfloat16),
            pltpu.SemaphoreType.DMA((3,)),
            pltpu.SemaphoreType.DMA((3,)),
        ],
        compiler_params=pltpu.CompilerParams(collective_id=0),
    )(x2d, wdkv, wuk, wuv)


def _mm_body(a_ref, w_ref, o_ref):
    o_ref[...] = jnp.dot(
        a_ref[...].astype(jnp.bfloat16), w_ref[...].astype(jnp.bfloat16),
        preferred_element_type=jnp.float32).astype(o_ref.dtype)


def _matmul(a, w, out_dtype, n_block):
    m, k = a.shape
    _, n = w.shape
    grid = n // n_block
    return pl.pallas_call(
        _mm_body,
        grid=(grid,),
        out_shape=jax.ShapeDtypeStruct((m, n), out_dtype),
        in_specs=[
            pl.BlockSpec((m, k), lambda j: (0, 0)),
            pl.BlockSpec((k, n_block), lambda j: (0, j)),
        ],
        out_specs=pl.BlockSpec((m, n_block), lambda j: (0, j)),
    )(a, w)


HG = 8


def _attn_body(q_ref, k_ref, v_ref, qr_ref, kr_ref, o_ref):
    kr = kr_ref[...]
    for h in range(HG):
        q = q_ref[:, h * Dh:(h + 1) * Dh]
        k = k_ref[:, h * Dh:(h + 1) * Dh]
        qr = qr_ref[:, h * Dr:(h + 1) * Dr]
        s = lax.dot_general(q, k, (((1,), (1,)), ((), ())),
                            preferred_element_type=jnp.float32)
        s = s + lax.dot_general(qr, kr, (((1,), (1,)), ((), ())),
                                preferred_element_type=jnp.float32)
        s = s * SCALE
        m_ = jnp.max(s, axis=1, keepdims=True)
        p = jnp.exp(s - m_)
        p = p / jnp.sum(p, axis=1, keepdims=True)
        o_ref[:, h * Dh:(h + 1) * Dh] = jnp.dot(
            p.astype(jnp.bfloat16), v_ref[:, h * Dh:(h + 1) * Dh],
            preferred_element_type=jnp.float32).astype(jnp.bfloat16)


def _attention(q2d, k2d, v2d, qr2d, kr2d):
    return pl.pallas_call(
        _attn_body,
        grid=(B, H // HG),
        out_shape=jax.ShapeDtypeStruct((M, H * Dh), jnp.bfloat16),
        in_specs=[
            pl.BlockSpec((S, HG * Dh), lambda b, g: (b, g)),
            pl.BlockSpec((S, HG * Dh), lambda b, g: (b, g)),
            pl.BlockSpec((S, HG * Dh), lambda b, g: (b, g)),
            pl.BlockSpec((S, HG * Dr), lambda b, g: (b, g)),
            pl.BlockSpec((S, Dr), lambda b, g: (b, 0)),
        ],
        out_specs=pl.BlockSpec((S, HG * Dh), lambda b, g: (b, g)),
    )(q2d, k2d, v2d, qr2d, kr2d)


def kernel(x, Wdkv, Wuk, Wuv, Wq, Wqr, Wkr, Wo):
    x2d = x.reshape(M, D)

    k2d, v2d = _kv_exchange(x2d, Wdkv, Wuk, Wuv)
    q2d = _matmul(x2d, Wq, jnp.bfloat16, 512)
    qr2d = _matmul(x2d, Wqr, jnp.bfloat16, 512)
    kr2d = _matmul(x2d, Wkr, jnp.bfloat16, 64)

    o2d = _attention(q2d, k2d, v2d, qr2d, kr2d)
    out = _matmul(o2d, Wo, jnp.float32, 512)
    return out.reshape(B, S, D)


# baseline (device time: 252215 ns/iter reference)
import functools

import jax
import jax.numpy as jnp
from jax import lax
from jax.experimental import pallas as pl
from jax.experimental.pallas import tpu as pltpu

B, S, H, Dh, Dr = 4, 256, 32, 128, 64
D = 4096
DC_SH = 128
M = B * S
SCALE = (Dh + Dr) ** -0.5
VMEM_LIMIT = 60 * 1024 * 1024


def _kv_body(x_ref, wdkv_ref, wuk_ref, wuv_ref,
             k_ref, v_ref,
             c_mine, c_other, wuk_other, wuv_other,
             send_sems, recv_sems):
    my_x = lax.axis_index("x")
    my_y = lax.axis_index("y")
    nbr = (my_x, 1 - my_y)

    barrier_sem = pltpu.get_barrier_semaphore()
    pl.semaphore_signal(barrier_sem, inc=1, device_id=nbr,
                        device_id_type=pl.DeviceIdType.MESH)
    pl.semaphore_wait(barrier_sem, 1)

    rdma_wuk = pltpu.make_async_remote_copy(
        src_ref=wuk_ref, dst_ref=wuk_other,
        send_sem=send_sems.at[0], recv_sem=recv_sems.at[0],
        device_id=nbr, device_id_type=pl.DeviceIdType.MESH)
    rdma_wuk.start()
    rdma_wuv = pltpu.make_async_remote_copy(
        src_ref=wuv_ref, dst_ref=wuv_other,
        send_sem=send_sems.at[1], recv_sem=recv_sems.at[1],
        device_id=nbr, device_id_type=pl.DeviceIdType.MESH)
    rdma_wuv.start()

    c_mine[...] = jnp.dot(
        x_ref[...], wdkv_ref[...],
        preferred_element_type=jnp.float32).astype(jnp.bfloat16)
    rdma_c = pltpu.make_async_remote_copy(
        src_ref=c_mine, dst_ref=c_other,
        send_sem=send_sems.at[2], recv_sem=recv_sems.at[2],
        device_id=nbr, device_id_type=pl.DeviceIdType.MESH)
    rdma_c.start()

    rdma_wuk.wait()
    rdma_wuv.wait()
    rdma_c.wait()

    NB = 1024
    for j in range(0, H * Dh, NB):
        sl = pl.ds(j, NB)
        k_ref[:, sl] = (
            jnp.dot(c_mine[...], wuk_ref[:, sl],
                    preferred_element_type=jnp.float32)
            + jnp.dot(c_other[...], wuk_other[:, sl],
                      preferred_element_type=jnp.float32)
        ).astype(jnp.bfloat16)
        v_ref[:, sl] = (
            jnp.dot(c_mine[...], wuv_ref[:, sl],
                    preferred_element_type=jnp.float32)
            + jnp.dot(c_other[...], wuv_other[:, sl],
                      preferred_element_type=jnp.float32)
        ).astype(jnp.bfloat16)


def _kv_exchange(x2d, wdkv, wuk, wuv):
    return pl.pallas_call(
        _kv_body,
        out_shape=[
            jax.ShapeDtypeStruct((M, H * Dh), jnp.bfloat16),
            jax.ShapeDtypeStruct((M, H * Dh), jnp.bfloat16),
        ],
        in_specs=[pl.BlockSpec(memory_space=pltpu.VMEM)] * 4,
        out_specs=[pl.BlockSpec(memory_space=pltpu.VMEM)] * 2,
        scratch_shapes=[
            pltpu.VMEM((M, DC_SH), jnp.bfloat16),
            pltpu.VMEM((M, DC_SH), jnp.bfloat16),
            pltpu.VMEM((DC_SH, H * Dh), jnp.bfloat16),
            pltpu.VMEM((DC_SH, H * Dh), jnp.bfloat16),
            pltpu.SemaphoreType.DMA((3,)),
            pltpu.SemaphoreType.DMA((3,)),
        ],
        compiler_params=pltpu.CompilerParams(
            collective_id=0, vmem_limit_bytes=VMEM_LIMIT),
    )(x2d, wdkv, wuk, wuv)


def _mm_body(a_ref, w_ref, o_ref):
    o_ref[...] = jnp.dot(
        a_ref[...], w_ref[...].astype(jnp.bfloat16),
        preferred_element_type=jnp.float32).astype(o_ref.dtype)


def _matmul(a, w, out_dtype, n_block):
    m, k = a.shape
    _, n = w.shape
    grid = n // n_block
    return pl.pallas_call(
        _mm_body,
        grid=(grid,),
        out_shape=jax.ShapeDtypeStruct((m, n), out_dtype),
        in_specs=[
            pl.BlockSpec((m, k), lambda j: (0, 0)),
            pl.BlockSpec((k, n_block), lambda j: (0, j)),
        ],
        out_specs=pl.BlockSpec((m, n_block), lambda j: (0, j)),
        compiler_params=pltpu.CompilerParams(vmem_limit_bytes=VMEM_LIMIT),
    )(a, w)


HG = 8


def _attn_body(q_ref, k_ref, v_ref, qr_ref, kr_ref, o_ref):
    kr = kr_ref[...]
    for h in range(HG):
        q = q_ref[:, h * Dh:(h + 1) * Dh]
        k = k_ref[:, h * Dh:(h + 1) * Dh]
        qr = qr_ref[:, h * Dr:(h + 1) * Dr]
        s = lax.dot_general(q, k, (((1,), (1,)), ((), ())),
                            preferred_element_type=jnp.float32)
        s = s + lax.dot_general(qr, kr, (((1,), (1,)), ((), ())),
                                preferred_element_type=jnp.float32)
        s = s * SCALE
        m_ = jnp.max(s, axis=1, keepdims=True)
        p = jnp.exp(s - m_)
        p = p / jnp.sum(p, axis=1, keepdims=True)
        o_ref[:, h * Dh:(h + 1) * Dh] = jnp.dot(
            p.astype(jnp.bfloat16), v_ref[:, h * Dh:(h + 1) * Dh],
            preferred_element_type=jnp.float32).astype(jnp.bfloat16)


def _attention(q2d, k2d, v2d, qr2d, kr2d):
    return pl.pallas_call(
        _attn_body,
        grid=(B, H // HG),
        out_shape=jax.ShapeDtypeStruct((M, H * Dh), jnp.bfloat16),
        in_specs=[
            pl.BlockSpec((S, HG * Dh), lambda b, g: (b, g)),
            pl.BlockSpec((S, HG * Dh), lambda b, g: (b, g)),
            pl.BlockSpec((S, HG * Dh), lambda b, g: (b, g)),
            pl.BlockSpec((S, HG * Dr), lambda b, g: (b, g)),
            pl.BlockSpec((S, Dr), lambda b, g: (b, 0)),
        ],
        out_specs=pl.BlockSpec((S, HG * Dh), lambda b, g: (b, g)),
        compiler_params=pltpu.CompilerParams(vmem_limit_bytes=VMEM_LIMIT),
    )(q2d, k2d, v2d, qr2d, kr2d)


def kernel(x, Wdkv, Wuk, Wuv, Wq, Wqr, Wkr, Wo):
    x2d = x.reshape(M, D).astype(jnp.bfloat16)
    wdkv = Wdkv.astype(jnp.bfloat16)
    wuk = Wuk.astype(jnp.bfloat16)
    wuv = Wuv.astype(jnp.bfloat16)

    k2d, v2d = _kv_exchange(x2d, wdkv, wuk, wuv)
    q2d = _matmul(x2d, Wq, jnp.bfloat16, 512)
    qr2d = _matmul(x2d, Wqr, jnp.bfloat16, 512)
    kr2d = _matmul(x2d, Wkr, jnp.bfloat16, 64)

    o2d = _attention(q2d, k2d, v2d, qr2d, kr2d)
    out = _matmul(o2d, Wo, jnp.float32, 512)
    return out.reshape(B, S, D)


# device time: 217507 ns/iter; 1.1596x vs baseline; 1.1596x over previous
import functools

import jax
import jax.numpy as jnp
from jax import lax
from jax.experimental import pallas as pl
from jax.experimental.pallas import tpu as pltpu

B, S, H, Dh, Dr = 4, 256, 32, 128, 64
D = 4096
DC_SH = 128
M = B * S
SCALE = (Dh + Dr) ** -0.5
VMEM_LIMIT = 60 * 1024 * 1024


def _kv_body(x_ref, wdkv_ref, wuk_ref, wuv_ref, wkr_ref,
             k_ref, v_ref, kr_ref,
             c_mine, c_other, wuk_other, wuv_other, acc,
             send_sems, recv_sems):
    my_x = lax.axis_index("x")
    my_y = lax.axis_index("y")
    nbr = (my_x, 1 - my_y)

    barrier_sem = pltpu.get_barrier_semaphore()
    pl.semaphore_signal(barrier_sem, inc=1, device_id=nbr,
                        device_id_type=pl.DeviceIdType.MESH)
    pl.semaphore_wait(barrier_sem, 1)

    c_mine[...] = jnp.dot(
        x_ref[...], wdkv_ref[...],
        preferred_element_type=jnp.float32).astype(jnp.bfloat16)
    rdma_c = pltpu.make_async_remote_copy(
        src_ref=c_mine, dst_ref=c_other,
        send_sem=send_sems.at[2], recv_sem=recv_sems.at[2],
        device_id=nbr, device_id_type=pl.DeviceIdType.MESH)
    rdma_c.start()
    rdma_wuk = pltpu.make_async_remote_copy(
        src_ref=wuk_ref, dst_ref=wuk_other,
        send_sem=send_sems.at[0], recv_sem=recv_sems.at[0],
        device_id=nbr, device_id_type=pl.DeviceIdType.MESH)
    rdma_wuk.start()
    rdma_wuv = pltpu.make_async_remote_copy(
        src_ref=wuv_ref, dst_ref=wuv_other,
        send_sem=send_sems.at[1], recv_sem=recv_sems.at[1],
        device_id=nbr, device_id_type=pl.DeviceIdType.MESH)
    rdma_wuv.start()

    kr_ref[...] = jnp.dot(
        x_ref[...], wkr_ref[...],
        preferred_element_type=jnp.float32).astype(jnp.bfloat16)
    acc[...] = jnp.dot(c_mine[...], wuk_ref[...],
                       preferred_element_type=jnp.float32)

    rdma_c.wait()
    rdma_wuk.wait()
    NB = 1024
    for j in range(0, H * Dh, NB):
        sl = pl.ds(j, NB)
        k_ref[:, sl] = (
            acc[:, sl]
            + jnp.dot(c_other[...], wuk_other[:, sl],
                      preferred_element_type=jnp.float32)
        ).astype(jnp.bfloat16)

    acc[...] = jnp.dot(c_mine[...], wuv_ref[...],
                       preferred_element_type=jnp.float32)
    rdma_wuv.wait()
    for j in range(0, H * Dh, NB):
        sl = pl.ds(j, NB)
        v_ref[:, sl] = (
            acc[:, sl]
            + jnp.dot(c_other[...], wuv_other[:, sl],
                      preferred_element_type=jnp.float32)
        ).astype(jnp.bfloat16)


def _kv_exchange(x2d, wdkv, wuk, wuv, wkr):
    return pl.pallas_call(
        _kv_body,
        out_shape=[
            jax.ShapeDtypeStruct((M, H * Dh), jnp.bfloat16),
            jax.ShapeDtypeStruct((M, H * Dh), jnp.bfloat16),
            jax.ShapeDtypeStruct((M, Dr), jnp.bfloat16),
        ],
        in_specs=[pl.BlockSpec(memory_space=pltpu.VMEM)] * 5,
        out_specs=[pl.BlockSpec(memory_space=pltpu.VMEM)] * 3,
        scratch_shapes=[
            pltpu.VMEM((M, DC_SH), jnp.bfloat16),
            pltpu.VMEM((M, DC_SH), jnp.bfloat16),
            pltpu.VMEM((DC_SH, H * Dh), jnp.bfloat16),
            pltpu.VMEM((DC_SH, H * Dh), jnp.bfloat16),
            pltpu.VMEM((M, H * Dh), jnp.float32),
            pltpu.SemaphoreType.DMA((3,)),
            pltpu.SemaphoreType.DMA((3,)),
        ],
        compiler_params=pltpu.CompilerParams(
            collective_id=0, vmem_limit_bytes=VMEM_LIMIT),
    )(x2d, wdkv, wuk, wuv, wkr)


def _mm_body(a_ref, w_ref, o_ref):
    o_ref[...] = jnp.dot(
        a_ref[...], w_ref[...].astype(jnp.bfloat16),
        preferred_element_type=jnp.float32).astype(o_ref.dtype)


def _matmul(a, w, out_dtype, n_block):
    m, k = a.shape
    _, n = w.shape
    grid = n // n_block
    return pl.pallas_call(
        _mm_body,
        grid=(grid,),
        out_shape=jax.ShapeDtypeStruct((m, n), out_dtype),
        in_specs=[
            pl.BlockSpec((m, k), lambda j: (0, 0)),
            pl.BlockSpec((k, n_block), lambda j: (0, j)),
        ],
        out_specs=pl.BlockSpec((m, n_block), lambda j: (0, j)),
        compiler_params=pltpu.CompilerParams(vmem_limit_bytes=VMEM_LIMIT),
    )(a, w)


HG = 8


def _attn_body(q_ref, k_ref, v_ref, qr_ref, kr_ref, o_ref):
    kr = kr_ref[...]
    scale = jnp.bfloat16(SCALE)
    for h in range(HG):
        q = q_ref[:, h * Dh:(h + 1) * Dh] * scale
        k = k_ref[:, h * Dh:(h + 1) * Dh]
        qr = qr_ref[:, h * Dr:(h + 1) * Dr] * scale
        s = lax.dot_general(q, k, (((1,), (1,)), ((), ())),
                            preferred_element_type=jnp.float32)
        s = s + lax.dot_general(qr, kr, (((1,), (1,)), ((), ())),
                                preferred_element_type=jnp.float32)
        p = jnp.exp(s)
        o = jnp.dot(p.astype(jnp.bfloat16), v_ref[:, h * Dh:(h + 1) * Dh],
                    preferred_element_type=jnp.float32)
        recip = 1.0 / jnp.sum(p, axis=1, keepdims=True)
        o_ref[:, h * Dh:(h + 1) * Dh] = (o * recip).astype(jnp.bfloat16)


def _attention(q2d, k2d, v2d, qr2d, kr2d):
    return pl.pallas_call(
        _attn_body,
        grid=(B, H // HG),
        out_shape=jax.ShapeDtypeStruct((M, H * Dh), jnp.bfloat16),
        in_specs=[
            pl.BlockSpec((S, HG * Dh), lambda b, g: (b, g)),
            pl.BlockSpec((S, HG * Dh), lambda b, g: (b, g)),
            pl.BlockSpec((S, HG * Dh), lambda b, g: (b, g)),
            pl.BlockSpec((S, HG * Dr), lambda b, g: (b, g)),
            pl.BlockSpec((S, Dr), lambda b, g: (b, 0)),
        ],
        out_specs=pl.BlockSpec((S, HG * Dh), lambda b, g: (b, g)),
        compiler_params=pltpu.CompilerParams(vmem_limit_bytes=VMEM_LIMIT),
    )(q2d, k2d, v2d, qr2d, kr2d)


def kernel(x, Wdkv, Wuk, Wuv, Wq, Wqr, Wkr, Wo):
    x2d = x.reshape(M, D).astype(jnp.bfloat16)
    wdkv = Wdkv.astype(jnp.bfloat16)
    wuk = Wuk.astype(jnp.bfloat16)
    wuv = Wuv.astype(jnp.bfloat16)
    wkr = Wkr.astype(jnp.bfloat16)

    k2d, v2d, kr2d = _kv_exchange(x2d, wdkv, wuk, wuv, wkr)
    q2d = _matmul(x2d, Wq, jnp.bfloat16, 512)
    qr2d = _matmul(x2d, Wqr, jnp.bfloat16, 512)

    o2d = _attention(q2d, k2d, v2d, qr2d, kr2d)
    out = _matmul(o2d, Wo, jnp.float32, 512)
    return out.reshape(B, S, D)


# device time: 216678 ns/iter; 1.1640x vs baseline; 1.0038x over previous
import functools

import jax
import jax.numpy as jnp
from jax import lax
from jax.experimental import pallas as pl
from jax.experimental.pallas import tpu as pltpu

B, S, H, Dh, Dr = 4, 256, 32, 128, 64
D = 4096
DC_SH = 128
M = B * S
SCALE = (Dh + Dr) ** -0.5
VMEM_LIMIT = 60 * 1024 * 1024


def _kv_body(x_ref, wdkv_ref, wuk_ref, wuv_ref, wkr_ref,
             k_ref, v_ref, kr_ref, xb_ref,
             c_buf, c_full, wuk_full, wuv_full,
             send_sems, recv_sems):
    my_x = lax.axis_index("x")
    my_y = lax.axis_index("y")
    nbr = (my_x, 1 - my_y)

    barrier_sem = pltpu.get_barrier_semaphore()
    pl.semaphore_signal(barrier_sem, inc=1, device_id=nbr,
                        device_id_type=pl.DeviceIdType.MESH)
    pl.semaphore_wait(barrier_sem, 1)

    xb_ref[...] = x_ref[...].astype(jnp.bfloat16)

    cm = jnp.dot(xb_ref[...], wdkv_ref[...],
                 preferred_element_type=jnp.float32).astype(jnp.bfloat16)

    def _start(slot):
        c_buf[slot] = cm
        row = slice(slot * DC_SH, (slot + 1) * DC_SH)
        wuk_full[row, :] = wuk_ref[...]
        wuv_full[row, :] = wuv_ref[...]
        pltpu.make_async_remote_copy(
            src_ref=c_buf.at[slot], dst_ref=c_buf.at[slot],
            send_sem=send_sems.at[2], recv_sem=recv_sems.at[2],
            device_id=nbr, device_id_type=pl.DeviceIdType.MESH).start()
        pltpu.make_async_remote_copy(
            src_ref=wuk_full.at[pl.ds(slot * DC_SH, DC_SH)],
            dst_ref=wuk_full.at[pl.ds(slot * DC_SH, DC_SH)],
            send_sem=send_sems.at[0], recv_sem=recv_sems.at[0],
            device_id=nbr, device_id_type=pl.DeviceIdType.MESH).start()
        pltpu.make_async_remote_copy(
            src_ref=wuv_full.at[pl.ds(slot * DC_SH, DC_SH)],
            dst_ref=wuv_full.at[pl.ds(slot * DC_SH, DC_SH)],
            send_sem=send_sems.at[1], recv_sem=recv_sems.at[1],
            device_id=nbr, device_id_type=pl.DeviceIdType.MESH).start()

    pl.when(my_y == 0)(lambda: _start(0))
    pl.when(my_y == 1)(lambda: _start(1))

    rdma_c = pltpu.make_async_remote_copy(
        src_ref=c_buf.at[0], dst_ref=c_buf.at[0],
        send_sem=send_sems.at[2], recv_sem=recv_sems.at[2],
        device_id=nbr, device_id_type=pl.DeviceIdType.MESH)
    rdma_wuk = pltpu.make_async_remote_copy(
        src_ref=wuk_full.at[pl.ds(0, DC_SH)],
        dst_ref=wuk_full.at[pl.ds(0, DC_SH)],
        send_sem=send_sems.at[0], recv_sem=recv_sems.at[0],
        device_id=nbr, device_id_type=pl.DeviceIdType.MESH)
    rdma_wuv = pltpu.make_async_remote_copy(
        src_ref=wuv_full.at[pl.ds(0, DC_SH)],
        dst_ref=wuv_full.at[pl.ds(0, DC_SH)],
        send_sem=send_sems.at[1], recv_sem=recv_sems.at[1],
        device_id=nbr, device_id_type=pl.DeviceIdType.MESH)

    kr_ref[...] = jnp.dot(
        xb_ref[...], wkr_ref[...],
        preferred_element_type=jnp.float32).astype(jnp.bfloat16)

    rdma_c.wait()
    rdma_wuk.wait()
    c_full[:, 0:DC_SH] = c_buf[0]
    c_full[:, DC_SH:2 * DC_SH] = c_buf[1]
    NB = 1024
    for j in range(0, H * Dh, NB):
        sl = pl.ds(j, NB)
        k_ref[:, sl] = jnp.dot(
            c_full[...], wuk_full[:, sl],
            preferred_element_type=jnp.float32).astype(jnp.bfloat16)
    rdma_wuv.wait()
    for j in range(0, H * Dh, NB):
        sl = pl.ds(j, NB)
        v_ref[:, sl] = jnp.dot(
            c_full[...], wuv_full[:, sl],
            preferred_element_type=jnp.float32).astype(jnp.bfloat16)


def _kv_exchange(x2d, wdkv, wuk, wuv, wkr):
    return pl.pallas_call(
        _kv_body,
        out_shape=[
            jax.ShapeDtypeStruct((M, H * Dh), jnp.bfloat16),
            jax.ShapeDtypeStruct((M, H * Dh), jnp.bfloat16),
            jax.ShapeDtypeStruct((M, Dr), jnp.bfloat16),
            jax.ShapeDtypeStruct((M, D), jnp.bfloat16),
        ],
        in_specs=[pl.BlockSpec(memory_space=pltpu.VMEM)] * 5,
        out_specs=[pl.BlockSpec(memory_space=pltpu.VMEM)] * 4,
        scratch_shapes=[
            pltpu.VMEM((2, M, DC_SH), jnp.bfloat16),
            pltpu.VMEM((M, 2 * DC_SH), jnp.bfloat16),
            pltpu.VMEM((2 * DC_SH, H * Dh), jnp.bfloat16),
            pltpu.VMEM((2 * DC_SH, H * Dh), jnp.bfloat16),
            pltpu.SemaphoreType.DMA((3,)),
            pltpu.SemaphoreType.DMA((3,)),
        ],
        compiler_params=pltpu.CompilerParams(
            collective_id=0, vmem_limit_bytes=VMEM_LIMIT),
    )(x2d, wdkv, wuk, wuv, wkr)


def _mm_body(a_ref, w_ref, o_ref):
    o_ref[...] = jnp.dot(
        a_ref[...], w_ref[...].astype(jnp.bfloat16),
        preferred_element_type=jnp.float32).astype(o_ref.dtype)


def _matmul(a, w, out_dtype, n_block):
    m, k = a.shape
    _, n = w.shape
    grid = n // n_block
    return pl.pallas_call(
        _mm_body,
        grid=(grid,),
        out_shape=jax.ShapeDtypeStruct((m, n), out_dtype),
        in_specs=[
            pl.BlockSpec((m, k), lambda j: (0, 0)),
            pl.BlockSpec((k, n_block), lambda j: (0, j)),
        ],
        out_specs=pl.BlockSpec((m, n_block), lambda j: (0, j)),
        compiler_params=pltpu.CompilerParams(vmem_limit_bytes=VMEM_LIMIT),
    )(a, w)


HG = 8


def _attn_body(q_ref, k_ref, v_ref, qr_ref, kr_ref, o_ref):
    kr = kr_ref[...]
    scale = jnp.bfloat16(SCALE)
    ones_m = jnp.ones((S, 128), jnp.bfloat16)
    for h in range(HG):
        q = q_ref[:, h * Dh:(h + 1) * Dh] * scale
        k = k_ref[:, h * Dh:(h + 1) * Dh]
        qr = qr_ref[:, h * Dr:(h + 1) * Dr] * scale
        s = lax.dot_general(q, k, (((1,), (1,)), ((), ())),
                            preferred_element_type=jnp.float32)
        s = s + lax.dot_general(qr, kr, (((1,), (1,)), ((), ())),
                                preferred_element_type=jnp.float32)
        p = jnp.exp(s.astype(jnp.bfloat16))
        o = jnp.dot(p, v_ref[:, h * Dh:(h + 1) * Dh],
                    preferred_element_type=jnp.float32)
        ssum = jnp.dot(p, ones_m, preferred_element_type=jnp.float32)
        o_ref[:, h * Dh:(h + 1) * Dh] = (
            o * (1.0 / ssum[:, 0:1])).astype(jnp.bfloat16)


def _attention(q2d, k2d, v2d, qr2d, kr2d):
    return pl.pallas_call(
        _attn_body,
        grid=(B, H // HG),
        out_shape=jax.ShapeDtypeStruct((M, H * Dh), jnp.bfloat16),
        in_specs=[
            pl.BlockSpec((S, HG * Dh), lambda b, g: (b, g)),
            pl.BlockSpec((S, HG * Dh), lambda b, g: (b, g)),
            pl.BlockSpec((S, HG * Dh), lambda b, g: (b, g)),
            pl.BlockSpec((S, HG * Dr), lambda b, g: (b, g)),
            pl.BlockSpec((S, Dr), lambda b, g: (b, 0)),
        ],
        out_specs=pl.BlockSpec((S, HG * Dh), lambda b, g: (b, g)),
        compiler_params=pltpu.CompilerParams(vmem_limit_bytes=VMEM_LIMIT),
    )(q2d, k2d, v2d, qr2d, kr2d)


def kernel(x, Wdkv, Wuk, Wuv, Wq, Wqr, Wkr, Wo):
    x2d = x.reshape(M, D)
    wdkv = Wdkv.astype(jnp.bfloat16)
    wuk = Wuk.astype(jnp.bfloat16)
    wuv = Wuv.astype(jnp.bfloat16)
    wkr = Wkr.astype(jnp.bfloat16)

    k2d, v2d, kr2d, xb = _kv_exchange(x2d, wdkv, wuk, wuv, wkr)
    q2d = _matmul(xb, Wq, jnp.bfloat16, 1024)
    qr2d = _matmul(xb, Wqr, jnp.bfloat16, 1024)

    o2d = _attention(q2d, k2d, v2d, qr2d, kr2d)
    out = _matmul(o2d, Wo, jnp.float32, 1024)
    return out.reshape(B, S, D)


# device time: 196903 ns/iter; 1.2809x vs baseline; 1.1004x over previous
import jax
import jax.numpy as jnp
from jax import lax
from jax.experimental import pallas as pl
from jax.experimental.pallas import tpu as pltpu

B, S, H, Dh, Dr = 4, 256, 32, 128, 64
D = 4096
DC_SH = 128
M = B * S
MH = M // 2
SCALE = (Dh + Dr) ** -0.5
VMEM_LIMIT = 60 * 1024 * 1024


def _kv_body(x_ref, wdkv_ref, wuk_ref, wuv_ref, wkr_ref,
             k_ref, v_ref, kr_ref, xb_ref,
             c_buf, c_full, wuk_full, wuv_full,
             send_sems, recv_sems):
    my_x = lax.axis_index("x")
    my_y = lax.axis_index("y")
    nbr = (my_x, 1 - my_y)

    barrier_sem = pltpu.get_barrier_semaphore()
    pl.semaphore_signal(barrier_sem, inc=1, device_id=nbr,
                        device_id_type=pl.DeviceIdType.MESH)
    pl.semaphore_wait(barrier_sem, 1)

    xb_ref[...] = x_ref[...].astype(jnp.bfloat16)

    cm = jnp.dot(xb_ref[...], wdkv_ref[...],
                 preferred_element_type=jnp.float32).astype(jnp.bfloat16)

    def _start(slot):
        c_buf[slot] = cm
        row = slice(slot * DC_SH, (slot + 1) * DC_SH)
        wuk_full[row, :] = wuk_ref[...]
        wuv_full[row, :] = wuv_ref[...]
        pltpu.make_async_remote_copy(
            src_ref=c_buf.at[slot], dst_ref=c_buf.at[slot],
            send_sem=send_sems.at[2], recv_sem=recv_sems.at[2],
            device_id=nbr, device_id_type=pl.DeviceIdType.MESH).start()
        pltpu.make_async_remote_copy(
            src_ref=wuk_full.at[pl.ds(slot * DC_SH, DC_SH)],
            dst_ref=wuk_full.at[pl.ds(slot * DC_SH, DC_SH)],
            send_sem=send_sems.at[0], recv_sem=recv_sems.at[0],
            device_id=nbr, device_id_type=pl.DeviceIdType.MESH).start()
        pltpu.make_async_remote_copy(
            src_ref=wuv_full.at[pl.ds(slot * DC_SH, DC_SH)],
            dst_ref=wuv_full.at[pl.ds(slot * DC_SH, DC_SH)],
            send_sem=send_sems.at[1], recv_sem=recv_sems.at[1],
            device_id=nbr, device_id_type=pl.DeviceIdType.MESH).start()

    pl.when(my_y == 0)(lambda: _start(0))
    pl.when(my_y == 1)(lambda: _start(1))

    rdma_c = pltpu.make_async_remote_copy(
        src_ref=c_buf.at[0], dst_ref=c_buf.at[0],
        send_sem=send_sems.at[2], recv_sem=recv_sems.at[2],
        device_id=nbr, device_id_type=pl.DeviceIdType.MESH)
    rdma_wuk = pltpu.make_async_remote_copy(
        src_ref=wuk_full.at[pl.ds(0, DC_SH)],
        dst_ref=wuk_full.at[pl.ds(0, DC_SH)],
        send_sem=send_sems.at[0], recv_sem=recv_sems.at[0],
        device_id=nbr, device_id_type=pl.DeviceIdType.MESH)
    rdma_wuv = pltpu.make_async_remote_copy(
        src_ref=wuv_full.at[pl.ds(0, DC_SH)],
        dst_ref=wuv_full.at[pl.ds(0, DC_SH)],
        send_sem=send_sems.at[1], recv_sem=recv_sems.at[1],
        device_id=nbr, device_id_type=pl.DeviceIdType.MESH)

    kr_ref[...] = jnp.dot(
        xb_ref[...], wkr_ref[...],
        preferred_element_type=jnp.float32).astype(jnp.bfloat16)

    rdma_c.wait()
    rdma_wuk.wait()
    c_full[:, 0:DC_SH] = c_buf[0]
    c_full[:, DC_SH:2 * DC_SH] = c_buf[1]
    NB = 1024
    for j in range(0, H * Dh, NB):
        sl = pl.ds(j, NB)
        k_ref[:, sl] = jnp.dot(
            c_full[...], wuk_full[:, sl],
            preferred_element_type=jnp.float32).astype(jnp.bfloat16)
    rdma_wuv.wait()
    for j in range(0, H * Dh, NB):
        sl = pl.ds(j, NB)
        v_ref[:, sl] = jnp.dot(
            c_full[...], wuv_full[:, sl],
            preferred_element_type=jnp.float32).astype(jnp.bfloat16)


def _kv_exchange(xh, wdkv, wuk, wuv, wkr):
    return pl.pallas_call(
        _kv_body,
        out_shape=[
            jax.ShapeDtypeStruct((MH, H * Dh), jnp.bfloat16),
            jax.ShapeDtypeStruct((MH, H * Dh), jnp.bfloat16),
            jax.ShapeDtypeStruct((MH, Dr), jnp.bfloat16),
            jax.ShapeDtypeStruct((MH, D), jnp.bfloat16),
        ],
        in_specs=[pl.BlockSpec(memory_space=pltpu.VMEM)] * 5,
        out_specs=[pl.BlockSpec(memory_space=pltpu.VMEM)] * 4,
        scratch_shapes=[
            pltpu.VMEM((2, MH, DC_SH), jnp.bfloat16),
            pltpu.VMEM((MH, 2 * DC_SH), jnp.bfloat16),
            pltpu.VMEM((2 * DC_SH, H * Dh), jnp.bfloat16),
            pltpu.VMEM((2 * DC_SH, H * Dh), jnp.bfloat16),
            pltpu.SemaphoreType.DMA((3,)),
            pltpu.SemaphoreType.DMA((3,)),
        ],
        compiler_params=pltpu.CompilerParams(
            collective_id=0, vmem_limit_bytes=VMEM_LIMIT),
    )(xh, wdkv, wuk, wuv, wkr)


def _mm_body(a_ref, w_ref, o_ref):
    o_ref[...] = jnp.dot(
        a_ref[...], w_ref[...].astype(jnp.bfloat16),
        preferred_element_type=jnp.float32).astype(o_ref.dtype)


def _matmul(a, w, out_dtype, n_block):
    m, k = a.shape
    _, n = w.shape
    grid = n // n_block
    return pl.pallas_call(
        _mm_body,
        grid=(grid,),
        out_shape=jax.ShapeDtypeStruct((m, n), out_dtype),
        in_specs=[
            pl.BlockSpec((m, k), lambda j: (0, 0)),
            pl.BlockSpec((k, n_block), lambda j: (0, j)),
        ],
        out_specs=pl.BlockSpec((m, n_block), lambda j: (0, j)),
        compiler_params=pltpu.CompilerParams(vmem_limit_bytes=VMEM_LIMIT),
    )(a, w)


HG = 8
BH = MH // S


def _attn_body(q_ref, k_ref, v_ref, qr_ref, kr_ref, o_ref):
    kr = kr_ref[...]
    scale = jnp.bfloat16(SCALE)
    ones_m = jnp.ones((S, 128), jnp.bfloat16)
    for h in range(HG):
        q = q_ref[:, h * Dh:(h + 1) * Dh] * scale
        k = k_ref[:, h * Dh:(h + 1) * Dh]
        qr = qr_ref[:, h * Dr:(h + 1) * Dr] * scale
        s = lax.dot_general(q, k, (((1,), (1,)), ((), ())),
                            preferred_element_type=jnp.float32)
        s = s + lax.dot_general(qr, kr, (((1,), (1,)), ((), ())),
                                preferred_element_type=jnp.float32)
        p = jnp.exp(s.astype(jnp.bfloat16))
        o = jnp.dot(p, v_ref[:, h * Dh:(h + 1) * Dh],
                    preferred_element_type=jnp.float32)
        ssum = jnp.dot(p, ones_m, preferred_element_type=jnp.float32)
        o_ref[:, h * Dh:(h + 1) * Dh] = (
            o * (1.0 / ssum[:, 0:1])).astype(jnp.bfloat16)


def _attention(q2d, k2d, v2d, qr2d, kr2d):
    return pl.pallas_call(
        _attn_body,
        grid=(BH, H // HG),
        out_shape=jax.ShapeDtypeStruct((MH, H * Dh), jnp.bfloat16),
        in_specs=[
            pl.BlockSpec((S, HG * Dh), lambda b, g: (b, g)),
            pl.BlockSpec((S, HG * Dh), lambda b, g: (b, g)),
            pl.BlockSpec((S, HG * Dh), lambda b, g: (b, g)),
            pl.BlockSpec((S, HG * Dr), lambda b, g: (b, g)),
            pl.BlockSpec((S, Dr), lambda b, g: (b, 0)),
        ],
        out_specs=pl.BlockSpec((S, HG * Dh), lambda b, g: (b, g)),
        compiler_params=pltpu.CompilerParams(vmem_limit_bytes=VMEM_LIMIT),
    )(q2d, k2d, v2d, qr2d, kr2d)


NC = 512
NCHUNK = D // NC


def _out_body(o_ref, wo_ref, out_ref,
              wo_buf, sbuf, rbuf, dma_sems, send_sems, recv_sems):
    my_x = lax.axis_index("x")
    my_y = lax.axis_index("y")
    nbr = (1 - my_x, my_y)

    barrier_sem = pltpu.get_barrier_semaphore()
    pl.semaphore_signal(barrier_sem, inc=1, device_id=nbr,
                        device_id_type=pl.DeviceIdType.MESH)
    pl.semaphore_wait(barrier_sem, 1)

    def wo_copy(j):
        return pltpu.make_async_copy(
            wo_ref.at[:, pl.ds(j * NC, NC)], wo_buf.at[j % 2],
            dma_sems.at[j % 2])

    wo_copy(0).start()
    sends = []
    for j in range(NCHUNK):
        if j + 1 < NCHUNK:
            wo_copy(j + 1).start()
        wo_copy(j).wait()
        oj = jnp.dot(o_ref[...], wo_buf[j % 2].astype(jnp.bfloat16),
                     preferred_element_type=jnp.float32)
        cols = slice(j * NC, (j + 1) * NC)
        pl.when(my_x == 0)(
            lambda oj=oj, cols=cols: out_ref.__setitem__(
                (slice(0, MH), cols), oj))
        pl.when(my_x == 1)(
            lambda oj=oj, cols=cols: out_ref.__setitem__(
                (slice(MH, M), cols), oj))
        sbuf[j] = oj.astype(jnp.bfloat16)
        rdma = pltpu.make_async_remote_copy(
            src_ref=sbuf.at[j], dst_ref=rbuf.at[j],
            send_sem=send_sems.at[j], recv_sem=recv_sems.at[j],
            device_id=nbr, device_id_type=pl.DeviceIdType.MESH)
        rdma.start()
        sends.append(rdma)

    for j in range(NCHUNK):
        sends[j].wait_recv()
        cols = slice(j * NC, (j + 1) * NC)
        pl.when(my_x == 0)(
            lambda j=j, cols=cols: out_ref.__setitem__(
                (slice(MH, M), cols), rbuf[j].astype(jnp.float32)))
        pl.when(my_x == 1)(
            lambda j=j, cols=cols: out_ref.__setitem__(
                (slice(0, MH), cols), rbuf[j].astype(jnp.float32)))
    for j in range(NCHUNK):
        sends[j].wait_send()


def _out_proj_gather(o2d, Wo):
    return pl.pallas_call(
        _out_body,
        out_shape=jax.ShapeDtypeStruct((M, D), jnp.float32),
        in_specs=[
            pl.BlockSpec(memory_space=pltpu.VMEM),
            pl.BlockSpec(memory_space=pltpu.MemorySpace.HBM),
        ],
        out_specs=pl.BlockSpec(memory_space=pltpu.VMEM),
        scratch_shapes=[
            pltpu.VMEM((2, D, NC), jnp.float32),
            pltpu.VMEM((NCHUNK, MH, NC), jnp.bfloat16),
            pltpu.VMEM((NCHUNK, MH, NC), jnp.bfloat16),
            pltpu.SemaphoreType.DMA((2,)),
            pltpu.SemaphoreType.DMA((NCHUNK,)),
            pltpu.SemaphoreType.DMA((NCHUNK,)),
        ],
        compiler_params=pltpu.CompilerParams(
            collective_id=1, vmem_limit_bytes=VMEM_LIMIT),
    )(o2d, Wo)


def kernel(x, Wdkv, Wuk, Wuv, Wq, Wqr, Wkr, Wo):
    my_x = lax.axis_index("x")
    x2d = x.reshape(M, D)
    xh = lax.dynamic_slice_in_dim(x2d, my_x * MH, MH, 0)
    wdkv = Wdkv.astype(jnp.bfloat16)
    wuk = Wuk.astype(jnp.bfloat16)
    wuv = Wuv.astype(jnp.bfloat16)
    wkr = Wkr.astype(jnp.bfloat16)

    k2d, v2d, kr2d, xb = _kv_exchange(xh, wdkv, wuk, wuv, wkr)
    q2d = _matmul(xb, Wq, jnp.bfloat16, 1024)
    qr2d = _matmul(xb, Wqr, jnp.bfloat16, 1024)

    o2d = _attention(q2d, k2d, v2d, qr2d, kr2d)
    out = _out_proj_gather(o2d, Wo)
    return out.reshape(B, S, D)


# device time: 173181 ns/iter; 1.4564x vs baseline; 1.1370x over previous
import jax
import jax.numpy as jnp
from jax import lax
from jax.experimental import pallas as pl
from jax.experimental.pallas import tpu as pltpu

B, S, H, Dh, Dr = 4, 256, 32, 128, 64
D = 4096
DC_SH = 128
M = B * S
MH = M // 2
SCALE = (Dh + Dr) ** -0.5
VMEM_LIMIT = 60 * 1024 * 1024


def _kv_body(x_ref, wdkv_ref, wuk_ref, wuv_ref, wkr_ref, wqr_ref,
             k_ref, v_ref, kr_ref, xb_ref, qr_ref,
             c_buf, c_full, wuk_full, wuv_full, wqr_buf,
             send_sems, recv_sems, qdma_sems):
    my_x = lax.axis_index("x")
    my_y = lax.axis_index("y")
    nbr = (my_x, 1 - my_y)

    barrier_sem = pltpu.get_barrier_semaphore()
    pl.semaphore_signal(barrier_sem, inc=1, device_id=nbr,
                        device_id_type=pl.DeviceIdType.MESH)
    pl.semaphore_wait(barrier_sem, 1)

    pl.when(my_x == 0)(
        lambda: xb_ref.__setitem__(
            ..., x_ref[0:MH, :].astype(jnp.bfloat16)))
    pl.when(my_x == 1)(
        lambda: xb_ref.__setitem__(
            ..., x_ref[MH:M, :].astype(jnp.bfloat16)))

    cm = jnp.dot(xb_ref[...], wdkv_ref[...],
                 preferred_element_type=jnp.float32).astype(jnp.bfloat16)

    def _start(slot):
        c_buf[slot] = cm
        row = slice(slot * DC_SH, (slot + 1) * DC_SH)
        wuk_full[row, :] = wuk_ref[...]
        wuv_full[row, :] = wuv_ref[...]
        pltpu.make_async_remote_copy(
            src_ref=c_buf.at[slot], dst_ref=c_buf.at[slot],
            send_sem=send_sems.at[2], recv_sem=recv_sems.at[2],
            device_id=nbr, device_id_type=pl.DeviceIdType.MESH).start()
        pltpu.make_async_remote_copy(
            src_ref=wuk_full.at[pl.ds(slot * DC_SH, DC_SH)],
            dst_ref=wuk_full.at[pl.ds(slot * DC_SH, DC_SH)],
            send_sem=send_sems.at[0], recv_sem=recv_sems.at[0],
            device_id=nbr, device_id_type=pl.DeviceIdType.MESH).start()
        pltpu.make_async_remote_copy(
            src_ref=wuv_full.at[pl.ds(slot * DC_SH, DC_SH)],
            dst_ref=wuv_full.at[pl.ds(slot * DC_SH, DC_SH)],
            send_sem=send_sems.at[1], recv_sem=recv_sems.at[1],
            device_id=nbr, device_id_type=pl.DeviceIdType.MESH).start()

    pl.when(my_y == 0)(lambda: _start(0))
    pl.when(my_y == 1)(lambda: _start(1))

    rdma_c = pltpu.make_async_remote_copy(
        src_ref=c_buf.at[0], dst_ref=c_buf.at[0],
        send_sem=send_sems.at[2], recv_sem=recv_sems.at[2],
        device_id=nbr, device_id_type=pl.DeviceIdType.MESH)
    rdma_wuk = pltpu.make_async_remote_copy(
        src_ref=wuk_full.at[pl.ds(0, DC_SH)],
        dst_ref=wuk_full.at[pl.ds(0, DC_SH)],
        send_sem=send_sems.at[0], recv_sem=recv_sems.at[0],
        device_id=nbr, device_id_type=pl.DeviceIdType.MESH)
    rdma_wuv = pltpu.make_async_remote_copy(
        src_ref=wuv_full.at[pl.ds(0, DC_SH)],
        dst_ref=wuv_full.at[pl.ds(0, DC_SH)],
        send_sem=send_sems.at[1], recv_sem=recv_sems.at[1],
        device_id=nbr, device_id_type=pl.DeviceIdType.MESH)

    kr_ref[...] = jnp.dot(
        xb_ref[...], wkr_ref[...],
        preferred_element_type=jnp.float32).astype(jnp.bfloat16)

    QC = 256
    nq = (H * Dr) // QC

    def qr_copy(j):
        return pltpu.make_async_copy(
            wqr_ref.at[:, pl.ds(j * QC, QC)], wqr_buf.at[j % 2],
            qdma_sems.at[j % 2])

    qr_copy(0).start()
    for j in range(nq):
        if j + 1 < nq:
            qr_copy(j + 1).start()
        qr_copy(j).wait()
        qr_ref[:, j * QC:(j + 1) * QC] = jnp.dot(
            xb_ref[...], wqr_buf[j % 2].astype(jnp.bfloat16),
            preferred_element_type=jnp.float32).astype(jnp.bfloat16)

    rdma_c.wait()
    rdma_wuk.wait()
    c_full[:, 0:DC_SH] = c_buf[0]
    c_full[:, DC_SH:2 * DC_SH] = c_buf[1]
    NB = 1024
    for j in range(0, H * Dh, NB):
        sl = pl.ds(j, NB)
        k_ref[:, sl] = jnp.dot(
            c_full[...], wuk_full[:, sl],
            preferred_element_type=jnp.float32).astype(jnp.bfloat16)
    rdma_wuv.wait()
    for j in range(0, H * Dh, NB):
        sl = pl.ds(j, NB)
        v_ref[:, sl] = jnp.dot(
            c_full[...], wuv_full[:, sl],
            preferred_element_type=jnp.float32).astype(jnp.bfloat16)


def _kv_exchange(x2d, wdkv, wuk, wuv, wkr, wqr):
    return pl.pallas_call(
        _kv_body,
        out_shape=[
            jax.ShapeDtypeStruct((MH, H * Dh), jnp.bfloat16),
            jax.ShapeDtypeStruct((MH, H * Dh), jnp.bfloat16),
            jax.ShapeDtypeStruct((MH, Dr), jnp.bfloat16),
            jax.ShapeDtypeStruct((MH, D), jnp.bfloat16),
            jax.ShapeDtypeStruct((MH, H * Dr), jnp.bfloat16),
        ],
        in_specs=[pl.BlockSpec(memory_space=pltpu.VMEM)] * 5
        + [pl.BlockSpec(memory_space=pltpu.MemorySpace.HBM)],
        out_specs=[pl.BlockSpec(memory_space=pltpu.VMEM)] * 5,
        scratch_shapes=[
            pltpu.VMEM((2, MH, DC_SH), jnp.bfloat16),
            pltpu.VMEM((MH, 2 * DC_SH), jnp.bfloat16),
            pltpu.VMEM((2 * DC_SH, H * Dh), jnp.bfloat16),
            pltpu.VMEM((2 * DC_SH, H * Dh), jnp.bfloat16),
            pltpu.VMEM((2, D, 256), jnp.float32),
            pltpu.SemaphoreType.DMA((3,)),
            pltpu.SemaphoreType.DMA((3,)),
            pltpu.SemaphoreType.DMA((2,)),
        ],
        compiler_params=pltpu.CompilerParams(
            collective_id=0, vmem_limit_bytes=VMEM_LIMIT),
    )(x2d, wdkv, wuk, wuv, wkr, wqr)


def _mm_body(a_ref, w_ref, o_ref):
    a = a_ref[...]
    half = o_ref.shape[1] // 2
    w1 = w_ref[:, 0:half].astype(jnp.bfloat16)
    w2 = w_ref[:, half:].astype(jnp.bfloat16)
    o_ref[:, 0:half] = jnp.dot(
        a, w1, preferred_element_type=jnp.float32).astype(o_ref.dtype)
    o_ref[:, half:] = jnp.dot(
        a, w2, preferred_element_type=jnp.float32).astype(o_ref.dtype)


def _matmul(a, w, out_dtype, n_block):
    m, k = a.shape
    _, n = w.shape
    grid = n // n_block
    return pl.pallas_call(
        _mm_body,
        grid=(grid,),
        out_shape=jax.ShapeDtypeStruct((m, n), out_dtype),
        in_specs=[
            pl.BlockSpec((m, k), lambda j: (0, 0)),
            pl.BlockSpec((k, n_block), lambda j: (0, j)),
        ],
        out_specs=pl.BlockSpec((m, n_block), lambda j: (0, j)),
        compiler_params=pltpu.CompilerParams(vmem_limit_bytes=VMEM_LIMIT),
    )(a, w)


HG = 8
BH = MH // S


def _attn_body(q_ref, k_ref, v_ref, qr_ref, kr_ref, o_ref):
    kr = kr_ref[...]
    scale = jnp.bfloat16(SCALE)
    ones_m = jnp.ones((S, 128), jnp.bfloat16)
    for h in range(HG):
        q = q_ref[:, h * Dh:(h + 1) * Dh] * scale
        k = k_ref[:, h * Dh:(h + 1) * Dh]
        qr = qr_ref[:, h * Dr:(h + 1) * Dr] * scale
        s = lax.dot_general(q, k, (((1,), (1,)), ((), ())),
                            preferred_element_type=jnp.float32)
        s = s + lax.dot_general(qr, kr, (((1,), (1,)), ((), ())),
                                preferred_element_type=jnp.float32)
        p = jnp.exp(s.astype(jnp.bfloat16))
        o = jnp.dot(p, v_ref[:, h * Dh:(h + 1) * Dh],
                    preferred_element_type=jnp.float32)
        ssum = jnp.dot(p, ones_m, preferred_element_type=jnp.float32)
        o_ref[:, h * Dh:(h + 1) * Dh] = (
            o * (1.0 / ssum[:, 0:1])).astype(jnp.bfloat16)


def _attention(q2d, k2d, v2d, qr2d, kr2d):
    return pl.pallas_call(
        _attn_body,
        grid=(BH, H // HG),
        out_shape=jax.ShapeDtypeStruct((MH, H * Dh), jnp.bfloat16),
        in_specs=[
            pl.BlockSpec((S, HG * Dh), lambda b, g: (b, g)),
            pl.BlockSpec((S, HG * Dh), lambda b, g: (b, g)),
            pl.BlockSpec((S, HG * Dh), lambda b, g: (b, g)),
            pl.BlockSpec((S, HG * Dr), lambda b, g: (b, g)),
            pl.BlockSpec((S, Dr), lambda b, g: (b, 0)),
        ],
        out_specs=pl.BlockSpec((S, HG * Dh), lambda b, g: (b, g)),
        compiler_params=pltpu.CompilerParams(vmem_limit_bytes=VMEM_LIMIT),
    )(q2d, k2d, v2d, qr2d, kr2d)


NC = 512
NCHUNK = D // NC


def _out_body(o_ref, wo_ref, out_ref,
              wo_buf, sbuf, rbuf, dma_sems, send_sems, recv_sems):
    my_x = lax.axis_index("x")
    my_y = lax.axis_index("y")
    nbr = (1 - my_x, my_y)

    barrier_sem = pltpu.get_barrier_semaphore()
    pl.semaphore_signal(barrier_sem, inc=1, device_id=nbr,
                        device_id_type=pl.DeviceIdType.MESH)
    pl.semaphore_wait(barrier_sem, 1)

    def wo_copy(j):
        return pltpu.make_async_copy(
            wo_ref.at[:, pl.ds(j * NC, NC)], wo_buf.at[j % 2],
            dma_sems.at[j % 2])

    wo_copy(0).start()
    sends = []
    for j in range(NCHUNK):
        if j + 1 < NCHUNK:
            wo_copy(j + 1).start()
        wo_copy(j).wait()
        oj = jnp.dot(o_ref[...], wo_buf[j % 2].astype(jnp.bfloat16),
                     preferred_element_type=jnp.float32)
        cols = slice(j * NC, (j + 1) * NC)
        pl.when(my_x == 0)(
            lambda oj=oj, cols=cols: out_ref.__setitem__(
                (slice(0, MH), cols), oj))
        pl.when(my_x == 1)(
            lambda oj=oj, cols=cols: out_ref.__setitem__(
                (slice(MH, M), cols), oj))
        sbuf[j] = oj.astype(jnp.bfloat16)
        rdma = pltpu.make_async_remote_copy(
            src_ref=sbuf.at[j], dst_ref=rbuf.at[j],
            send_sem=send_sems.at[j], recv_sem=recv_sems.at[j],
            device_id=nbr, device_id_type=pl.DeviceIdType.MESH)
        rdma.start()
        sends.append(rdma)

    for j in range(NCHUNK):
        sends[j].wait_recv()
        cols = slice(j * NC, (j + 1) * NC)
        pl.when(my_x == 0)(
            lambda j=j, cols=cols: out_ref.__setitem__(
                (slice(MH, M), cols), rbuf[j].astype(jnp.float32)))
        pl.when(my_x == 1)(
            lambda j=j, cols=cols: out_ref.__setitem__(
                (slice(0, MH), cols), rbuf[j].astype(jnp.float32)))
    for j in range(NCHUNK):
        sends[j].wait_send()


def _out_proj_gather(o2d, Wo):
    return pl.pallas_call(
        _out_body,
        out_shape=jax.ShapeDtypeStruct((M, D), jnp.float32),
        in_specs=[
            pl.BlockSpec(memory_space=pltpu.VMEM),
            pl.BlockSpec(memory_space=pltpu.MemorySpace.HBM),
        ],
        out_specs=pl.BlockSpec(memory_space=pltpu.VMEM),
        scratch_shapes=[
            pltpu.VMEM((2, D, NC), jnp.float32),
            pltpu.VMEM((NCHUNK, MH, NC), jnp.bfloat16),
            pltpu.VMEM((NCHUNK, MH, NC), jnp.bfloat16),
            pltpu.SemaphoreType.DMA((2,)),
            pltpu.SemaphoreType.DMA((NCHUNK,)),
            pltpu.SemaphoreType.DMA((NCHUNK,)),
        ],
        compiler_params=pltpu.CompilerParams(
            collective_id=1, vmem_limit_bytes=VMEM_LIMIT),
    )(o2d, Wo)


def kernel(x, Wdkv, Wuk, Wuv, Wq, Wqr, Wkr, Wo):
    x2d = x.reshape(M, D)
    wdkv = Wdkv.astype(jnp.bfloat16)
    wuk = Wuk.astype(jnp.bfloat16)
    wuv = Wuv.astype(jnp.bfloat16)
    wkr = Wkr.astype(jnp.bfloat16)

    k2d, v2d, kr2d, xb, qr2d = _kv_exchange(x2d, wdkv, wuk, wuv, wkr, Wqr)
    q2d = _matmul(xb, Wq, jnp.bfloat16, 1024)

    o2d = _attention(q2d, k2d, v2d, qr2d, kr2d)
    out = _out_proj_gather(o2d, Wo)
    return out.reshape(B, S, D)


# device time: 171940 ns/iter; 1.4669x vs baseline; 1.0072x over previous
import jax
import jax.numpy as jnp
from jax import lax
from jax.experimental import pallas as pl
from jax.experimental.pallas import tpu as pltpu

B, S, H, Dh, Dr = 4, 256, 32, 128, 64
D = 4096
DC_SH = 128
M = B * S
MH = M // 2
SCALE = (Dh + Dr) ** -0.5
VMEM_LIMIT = 60 * 1024 * 1024


def _kv_body(x_ref, wdkv_ref, wuk_ref, wuv_ref, wkr_ref, wqr_ref,
             k_ref, v_ref, kr_ref, xb_ref, qr_ref,
             c_buf, c_full, wuk_full, wuv_full, wqr_buf,
             send_sems, recv_sems, qdma_sems):
    my_x = lax.axis_index("x")
    my_y = lax.axis_index("y")
    nbr = (my_x, 1 - my_y)

    barrier_sem = pltpu.get_barrier_semaphore()
    pl.semaphore_signal(barrier_sem, inc=1, device_id=nbr,
                        device_id_type=pl.DeviceIdType.MESH)
    pl.semaphore_wait(barrier_sem, 1)

    pl.when(my_x == 0)(
        lambda: xb_ref.__setitem__(
            ..., x_ref[0:MH, :].astype(jnp.bfloat16)))
    pl.when(my_x == 1)(
        lambda: xb_ref.__setitem__(
            ..., x_ref[MH:M, :].astype(jnp.bfloat16)))

    cm = jnp.dot(xb_ref[...], wdkv_ref[...],
                 preferred_element_type=jnp.float32).astype(jnp.bfloat16)

    def _start(slot):
        c_buf[slot] = cm
        row = slice(slot * DC_SH, (slot + 1) * DC_SH)
        wuk_full[row, :] = wuk_ref[...]
        wuv_full[row, :] = wuv_ref[...]
        pltpu.make_async_remote_copy(
            src_ref=c_buf.at[slot], dst_ref=c_buf.at[slot],
            send_sem=send_sems.at[2], recv_sem=recv_sems.at[2],
            device_id=nbr, device_id_type=pl.DeviceIdType.MESH).start()
        pltpu.make_async_remote_copy(
            src_ref=wuk_full.at[pl.ds(slot * DC_SH, DC_SH)],
            dst_ref=wuk_full.at[pl.ds(slot * DC_SH, DC_SH)],
            send_sem=send_sems.at[0], recv_sem=recv_sems.at[0],
            device_id=nbr, device_id_type=pl.DeviceIdType.MESH).start()
        pltpu.make_async_remote_copy(
            src_ref=wuv_full.at[pl.ds(slot * DC_SH, DC_SH)],
            dst_ref=wuv_full.at[pl.ds(slot * DC_SH, DC_SH)],
            send_sem=send_sems.at[1], recv_sem=recv_sems.at[1],
            device_id=nbr, device_id_type=pl.DeviceIdType.MESH).start()

    pl.when(my_y == 0)(lambda: _start(0))
    pl.when(my_y == 1)(lambda: _start(1))

    rdma_c = pltpu.make_async_remote_copy(
        src_ref=c_buf.at[0], dst_ref=c_buf.at[0],
        send_sem=send_sems.at[2], recv_sem=recv_sems.at[2],
        device_id=nbr, device_id_type=pl.DeviceIdType.MESH)
    rdma_wuk = pltpu.make_async_remote_copy(
        src_ref=wuk_full.at[pl.ds(0, DC_SH)],
        dst_ref=wuk_full.at[pl.ds(0, DC_SH)],
        send_sem=send_sems.at[0], recv_sem=recv_sems.at[0],
        device_id=nbr, device_id_type=pl.DeviceIdType.MESH)
    rdma_wuv = pltpu.make_async_remote_copy(
        src_ref=wuv_full.at[pl.ds(0, DC_SH)],
        dst_ref=wuv_full.at[pl.ds(0, DC_SH)],
        send_sem=send_sems.at[1], recv_sem=recv_sems.at[1],
        device_id=nbr, device_id_type=pl.DeviceIdType.MESH)

    kr_ref[...] = jnp.dot(
        xb_ref[...], wkr_ref[...],
        preferred_element_type=jnp.float32).astype(jnp.bfloat16)

    QC = 256
    nq = (H * Dr) // QC

    def qr_copy(j):
        return pltpu.make_async_copy(
            wqr_ref.at[:, pl.ds(j * QC, QC)], wqr_buf.at[j % 2],
            qdma_sems.at[j % 2])

    qr_copy(0).start()
    for j in range(nq):
        if j + 1 < nq:
            qr_copy(j + 1).start()
        qr_copy(j).wait()
        qr_ref[:, j * QC:(j + 1) * QC] = jnp.dot(
            xb_ref[...], wqr_buf[j % 2].astype(jnp.bfloat16),
            preferred_element_type=jnp.float32).astype(jnp.bfloat16)

    rdma_c.wait()
    rdma_wuk.wait()
    c_full[:, 0:DC_SH] = c_buf[0]
    c_full[:, DC_SH:2 * DC_SH] = c_buf[1]
    NB = 1024
    for j in range(0, H * Dh, NB):
        sl = pl.ds(j, NB)
        k_ref[:, sl] = jnp.dot(
            c_full[...], wuk_full[:, sl],
            preferred_element_type=jnp.float32).astype(jnp.bfloat16)
    rdma_wuv.wait()
    for j in range(0, H * Dh, NB):
        sl = pl.ds(j, NB)
        v_ref[:, sl] = jnp.dot(
            c_full[...], wuv_full[:, sl],
            preferred_element_type=jnp.float32).astype(jnp.bfloat16)


def _kv_exchange(x2d, wdkv, wuk, wuv, wkr, wqr):
    return pl.pallas_call(
        _kv_body,
        out_shape=[
            jax.ShapeDtypeStruct((MH, H * Dh), jnp.bfloat16),
            jax.ShapeDtypeStruct((MH, H * Dh), jnp.bfloat16),
            jax.ShapeDtypeStruct((MH, Dr), jnp.bfloat16),
            jax.ShapeDtypeStruct((MH, D), jnp.bfloat16),
            jax.ShapeDtypeStruct((MH, H * Dr), jnp.bfloat16),
        ],
        in_specs=[pl.BlockSpec(memory_space=pltpu.VMEM)] * 5
        + [pl.BlockSpec(memory_space=pltpu.MemorySpace.HBM)],
        out_specs=[pl.BlockSpec(memory_space=pltpu.VMEM)] * 5,
        scratch_shapes=[
            pltpu.VMEM((2, MH, DC_SH), jnp.bfloat16),
            pltpu.VMEM((MH, 2 * DC_SH), jnp.bfloat16),
            pltpu.VMEM((2 * DC_SH, H * Dh), jnp.bfloat16),
            pltpu.VMEM((2 * DC_SH, H * Dh), jnp.bfloat16),
            pltpu.VMEM((2, D, 256), jnp.float32),
            pltpu.SemaphoreType.DMA((3,)),
            pltpu.SemaphoreType.DMA((3,)),
            pltpu.SemaphoreType.DMA((2,)),
        ],
        compiler_params=pltpu.CompilerParams(
            collective_id=0, vmem_limit_bytes=VMEM_LIMIT),
    )(x2d, wdkv, wuk, wuv, wkr, wqr)


def _mm_body(a_ref, w_ref, o_ref):
    a = a_ref[...]
    half = o_ref.shape[1] // 2
    w1 = w_ref[:, 0:half].astype(jnp.bfloat16)
    w2 = w_ref[:, half:].astype(jnp.bfloat16)
    o_ref[:, 0:half] = jnp.dot(
        a, w1, preferred_element_type=jnp.float32).astype(o_ref.dtype)
    o_ref[:, half:] = jnp.dot(
        a, w2, preferred_element_type=jnp.float32).astype(o_ref.dtype)


def _matmul(a, w, out_dtype, n_block):
    m, k = a.shape
    _, n = w.shape
    grid = n // n_block
    return pl.pallas_call(
        _mm_body,
        grid=(grid,),
        out_shape=jax.ShapeDtypeStruct((m, n), out_dtype),
        in_specs=[
            pl.BlockSpec((m, k), lambda j: (0, 0)),
            pl.BlockSpec((k, n_block), lambda j: (0, j)),
        ],
        out_specs=pl.BlockSpec((m, n_block), lambda j: (0, j)),
        compiler_params=pltpu.CompilerParams(vmem_limit_bytes=VMEM_LIMIT),
    )(a, w)


HG = 8
BH = MH // S


def _attn_body(q_ref, k_ref, v_ref, qr_ref, kr_ref, o_ref):
    kr = kr_ref[...]
    scale = jnp.bfloat16(SCALE)
    ones_m = jnp.ones((S, 128), jnp.bfloat16)
    for h in range(HG):
        q = q_ref[:, h * Dh:(h + 1) * Dh] * scale
        k = k_ref[:, h * Dh:(h + 1) * Dh]
        qr = qr_ref[:, h * Dr:(h + 1) * Dr] * scale
        s = lax.dot_general(q, k, (((1,), (1,)), ((), ())),
                            preferred_element_type=jnp.float32)
        s = s + lax.dot_general(qr, kr, (((1,), (1,)), ((), ())),
                                preferred_element_type=jnp.float32)
        p = jnp.exp(s.astype(jnp.bfloat16))
        o = jnp.dot(p, v_ref[:, h * Dh:(h + 1) * Dh],
                    preferred_element_type=jnp.float32)
        ssum = jnp.dot(p, ones_m, preferred_element_type=jnp.float32)
        o_ref[:, h * Dh:(h + 1) * Dh] = (
            o * (1.0 / ssum[:, 0:1])).astype(jnp.bfloat16)


def _attention(q2d, k2d, v2d, qr2d, kr2d):
    return pl.pallas_call(
        _attn_body,
        grid=(BH, H // HG),
        out_shape=jax.ShapeDtypeStruct((MH, H * Dh), jnp.bfloat16),
        in_specs=[
            pl.BlockSpec((S, HG * Dh), lambda b, g: (b, g)),
            pl.BlockSpec((S, HG * Dh), lambda b, g: (b, g)),
            pl.BlockSpec((S, HG * Dh), lambda b, g: (b, g)),
            pl.BlockSpec((S, HG * Dr), lambda b, g: (b, g)),
            pl.BlockSpec((S, Dr), lambda b, g: (b, 0)),
        ],
        out_specs=pl.BlockSpec((S, HG * Dh), lambda b, g: (b, g)),
        compiler_params=pltpu.CompilerParams(vmem_limit_bytes=VMEM_LIMIT),
    )(q2d, k2d, v2d, qr2d, kr2d)


NC = 512
NCHUNK = D // NC


def _out_body(o_ref, wo_ref, out_ref,
              wo_buf, sbuf, rbufx, rbufy,
              dma_sems, sx_send, sx_recv, sy_send, sy_recv):
    my_x = lax.axis_index("x")
    my_y = lax.axis_index("y")
    xn = (1 - my_x, my_y)
    yn = (my_x, 1 - my_y)

    barrier_sem = pltpu.get_barrier_semaphore()
    for nbr in (xn, yn):
        pl.semaphore_signal(barrier_sem, inc=1, device_id=nbr,
                            device_id_type=pl.DeviceIdType.MESH)
    pl.semaphore_wait(barrier_sem, 2)

    def wo_copy(c, slot):
        return pltpu.make_async_copy(
            wo_ref.at[:, pl.ds(c * NC, NC)], wo_buf.at[slot],
            dma_sems.at[slot])

    def x_rdma(s):
        return pltpu.make_async_remote_copy(
            src_ref=sbuf.at[s], dst_ref=rbufx.at[s],
            send_sem=sx_send.at[s], recv_sem=sx_recv.at[s],
            device_id=xn, device_id_type=pl.DeviceIdType.MESH)

    def y_rdma(s):
        return pltpu.make_async_remote_copy(
            src_ref=rbufx.at[s], dst_ref=rbufy.at[s],
            send_sem=sy_send.at[s], recv_sem=sy_recv.at[s],
            device_id=yn, device_id_type=pl.DeviceIdType.MESH)

    def store(xv, yv, b0, c, val):
        pl.when(jnp.logical_and(my_x == xv, my_y == yv))(
            lambda: out_ref.__setitem__(
                (slice(b0, b0 + 2), slice(None),
                 slice(c * NC, (c + 1) * NC)),
                val.reshape(2, S, NC)))

    ORDER = [0, 4, 1, 5, 2, 6, 3, 7]
    wo_copy(ORDER[0], 0).start()
    for i, c in enumerate(ORDER):
        if i + 1 < NCHUNK:
            wo_copy(ORDER[i + 1], (i + 1) % 2).start()
        wo_copy(c, i % 2).wait()
        oj = jnp.dot(o_ref[...], wo_buf[i % 2].astype(jnp.bfloat16),
                     preferred_element_type=jnp.float32)
        for xv in (0, 1):
            for yv in (0, 1):
                store(xv, yv, 2 * xv, c, oj)
        s = c % 4
        cond = (my_y == 0) if c < 4 else (my_y == 1)

        def _send(s=s, oj=oj):
            sbuf[s] = oj.astype(jnp.bfloat16)
            x_rdma(s).start()

        pl.when(cond)(_send)

    for s in range(4):
        x_rdma(s).wait_recv()
        y_rdma(s).start()
        val = rbufx[s].astype(jnp.float32)
        for xv in (0, 1):
            for yv in (0, 1):
                store(xv, yv, 2 * (1 - xv), 4 * yv + s, val)
    for s in range(4):
        y_rdma(s).wait_recv()
        val = rbufy[s].astype(jnp.float32)
        for xv in (0, 1):
            for yv in (0, 1):
                store(xv, yv, 2 * (1 - xv), 4 * (1 - yv) + s, val)
    for s in range(4):
        x_rdma(s).wait_send()
        y_rdma(s).wait_send()


def _out_proj_gather(o2d, Wo):
    return pl.pallas_call(
        _out_body,
        out_shape=jax.ShapeDtypeStruct((B, S, D), jnp.float32),
        in_specs=[
            pl.BlockSpec(memory_space=pltpu.VMEM),
            pl.BlockSpec(memory_space=pltpu.MemorySpace.HBM),
        ],
        out_specs=pl.BlockSpec(memory_space=pltpu.VMEM),
        scratch_shapes=[
            pltpu.VMEM((2, D, NC), jnp.float32),
            pltpu.VMEM((4, MH, NC), jnp.bfloat16),
            pltpu.VMEM((4, MH, NC), jnp.bfloat16),
            pltpu.VMEM((4, MH, NC), jnp.bfloat16),
            pltpu.SemaphoreType.DMA((2,)),
            pltpu.SemaphoreType.DMA((4,)),
            pltpu.SemaphoreType.DMA((4,)),
            pltpu.SemaphoreType.DMA((4,)),
            pltpu.SemaphoreType.DMA((4,)),
        ],
        compiler_params=pltpu.CompilerParams(
            collective_id=1, vmem_limit_bytes=VMEM_LIMIT),
    )(o2d, Wo)


def kernel(x, Wdkv, Wuk, Wuv, Wq, Wqr, Wkr, Wo):
    x2d = x.reshape(M, D)
    wdkv = Wdkv.astype(jnp.bfloat16)
    wuk = Wuk.astype(jnp.bfloat16)
    wuv = Wuv.astype(jnp.bfloat16)
    wkr = Wkr.astype(jnp.bfloat16)

    k2d, v2d, kr2d, xb, qr2d = _kv_exchange(x2d, wdkv, wuk, wuv, wkr, Wqr)
    q2d = _matmul(xb, Wq, jnp.bfloat16, 1024)

    o2d = _attention(q2d, k2d, v2d, qr2d, kr2d)
    return _out_proj_gather(o2d, Wo)


# device time: 164419 ns/iter; 1.5340x vs baseline; 1.0457x over previous
import jax
import jax.numpy as jnp
from jax import lax
from jax.experimental import pallas as pl
from jax.experimental.pallas import tpu as pltpu

B, S, H, Dh, Dr = 4, 256, 32, 128, 64
D = 4096
DC_SH = 128
M = B * S
MH = M // 2
SCALE = (Dh + Dr) ** -0.5
VMEM_LIMIT = 60 * 1024 * 1024


def _kv_body(x_ref, wdkv_ref, wuk_ref, wuv_ref, wkr_ref, wqr_ref,
             k_ref, v_ref, kr_ref, xb_ref, qr_ref,
             c_buf, c_full, wuk_full, wuv_full, wqr_buf,
             send_sems, recv_sems, qdma_sems):
    my_x = lax.axis_index("x")
    my_y = lax.axis_index("y")
    nbr = (my_x, 1 - my_y)

    barrier_sem = pltpu.get_barrier_semaphore()
    pl.semaphore_signal(barrier_sem, inc=1, device_id=nbr,
                        device_id_type=pl.DeviceIdType.MESH)
    pl.semaphore_wait(barrier_sem, 1)

    pl.when(my_x == 0)(
        lambda: xb_ref.__setitem__(
            ..., x_ref[0:2].reshape(MH, D).astype(jnp.bfloat16)))
    pl.when(my_x == 1)(
        lambda: xb_ref.__setitem__(
            ..., x_ref[2:4].reshape(MH, D).astype(jnp.bfloat16)))

    cm = jnp.dot(xb_ref[...], wdkv_ref[...],
                 preferred_element_type=jnp.float32).astype(jnp.bfloat16)

    def _start(slot):
        c_buf[slot] = cm
        row = slice(slot * DC_SH, (slot + 1) * DC_SH)
        wuk_full[row, :] = wuk_ref[...]
        wuv_full[row, :] = wuv_ref[...]
        pltpu.make_async_remote_copy(
            src_ref=c_buf.at[slot], dst_ref=c_buf.at[slot],
            send_sem=send_sems.at[2], recv_sem=recv_sems.at[2],
            device_id=nbr, device_id_type=pl.DeviceIdType.MESH).start()
        pltpu.make_async_remote_copy(
            src_ref=wuk_full.at[pl.ds(slot * DC_SH, DC_SH)],
            dst_ref=wuk_full.at[pl.ds(slot * DC_SH, DC_SH)],
            send_sem=send_sems.at[0], recv_sem=recv_sems.at[0],
            device_id=nbr, device_id_type=pl.DeviceIdType.MESH).start()
        pltpu.make_async_remote_copy(
            src_ref=wuv_full.at[pl.ds(slot * DC_SH, DC_SH)],
            dst_ref=wuv_full.at[pl.ds(slot * DC_SH, DC_SH)],
            send_sem=send_sems.at[1], recv_sem=recv_sems.at[1],
            device_id=nbr, device_id_type=pl.DeviceIdType.MESH).start()

    pl.when(my_y == 0)(lambda: _start(0))
    pl.when(my_y == 1)(lambda: _start(1))

    rdma_c = pltpu.make_async_remote_copy(
        src_ref=c_buf.at[0], dst_ref=c_buf.at[0],
        send_sem=send_sems.at[2], recv_sem=recv_sems.at[2],
        device_id=nbr, device_id_type=pl.DeviceIdType.MESH)
    rdma_wuk = pltpu.make_async_remote_copy(
        src_ref=wuk_full.at[pl.ds(0, DC_SH)],
        dst_ref=wuk_full.at[pl.ds(0, DC_SH)],
        send_sem=send_sems.at[0], recv_sem=recv_sems.at[0],
        device_id=nbr, device_id_type=pl.DeviceIdType.MESH)
    rdma_wuv = pltpu.make_async_remote_copy(
        src_ref=wuv_full.at[pl.ds(0, DC_SH)],
        dst_ref=wuv_full.at[pl.ds(0, DC_SH)],
        send_sem=send_sems.at[1], recv_sem=recv_sems.at[1],
        device_id=nbr, device_id_type=pl.DeviceIdType.MESH)

    kr_ref[...] = jnp.dot(
        xb_ref[...], wkr_ref[...],
        preferred_element_type=jnp.float32).astype(jnp.bfloat16)

    QC = 256
    nq = (H * Dr) // QC

    def qr_copy(j):
        return pltpu.make_async_copy(
            wqr_ref.at[:, pl.ds(j * QC, QC)], wqr_buf.at[j % 2],
            qdma_sems.at[j % 2])

    qr_copy(0).start()
    for j in range(nq):
        if j + 1 < nq:
            qr_copy(j + 1).start()
        qr_copy(j).wait()
        qr_ref[:, j * QC:(j + 1) * QC] = jnp.dot(
            xb_ref[...], wqr_buf[j % 2].astype(jnp.bfloat16),
            preferred_element_type=jnp.float32).astype(jnp.bfloat16)

    rdma_c.wait()
    rdma_wuk.wait()
    c_full[:, 0:DC_SH] = c_buf[0]
    c_full[:, DC_SH:2 * DC_SH] = c_buf[1]
    NB = 1024
    for j in range(0, H * Dh, NB):
        sl = pl.ds(j, NB)
        k_ref[:, sl] = jnp.dot(
            c_full[...], wuk_full[:, sl],
            preferred_element_type=jnp.float32).astype(jnp.bfloat16)
    rdma_wuv.wait()
    for j in range(0, H * Dh, NB):
        sl = pl.ds(j, NB)
        v_ref[:, sl] = jnp.dot(
            c_full[...], wuv_full[:, sl],
            preferred_element_type=jnp.float32).astype(jnp.bfloat16)


def _kv_exchange(x2d, wdkv, wuk, wuv, wkr, wqr):
    return pl.pallas_call(
        _kv_body,
        out_shape=[
            jax.ShapeDtypeStruct((MH, H * Dh), jnp.bfloat16),
            jax.ShapeDtypeStruct((MH, H * Dh), jnp.bfloat16),
            jax.ShapeDtypeStruct((MH, Dr), jnp.bfloat16),
            jax.ShapeDtypeStruct((MH, D), jnp.bfloat16),
            jax.ShapeDtypeStruct((MH, H * Dr), jnp.bfloat16),
        ],
        in_specs=[pl.BlockSpec(memory_space=pltpu.VMEM)] * 5
        + [pl.BlockSpec(memory_space=pltpu.MemorySpace.HBM)],
        out_specs=[pl.BlockSpec(memory_space=pltpu.VMEM)] * 5,
        scratch_shapes=[
            pltpu.VMEM((2, MH, DC_SH), jnp.bfloat16),
            pltpu.VMEM((MH, 2 * DC_SH), jnp.bfloat16),
            pltpu.VMEM((2 * DC_SH, H * Dh), jnp.bfloat16),
            pltpu.VMEM((2 * DC_SH, H * Dh), jnp.bfloat16),
            pltpu.VMEM((2, D, 256), jnp.float32),
            pltpu.SemaphoreType.DMA((3,)),
            pltpu.SemaphoreType.DMA((3,)),
            pltpu.SemaphoreType.DMA((2,)),
        ],
        compiler_params=pltpu.CompilerParams(
            collective_id=0, vmem_limit_bytes=VMEM_LIMIT),
    )(x2d, wdkv, wuk, wuv, wkr, wqr)


def _mm_body(a_ref, w_ref, o_ref):
    a = a_ref[...]
    half = o_ref.shape[1] // 2
    w1 = w_ref[:, 0:half].astype(jnp.bfloat16)
    w2 = w_ref[:, half:].astype(jnp.bfloat16)
    o_ref[:, 0:half] = jnp.dot(
        a, w1, preferred_element_type=jnp.float32).astype(o_ref.dtype)
    o_ref[:, half:] = jnp.dot(
        a, w2, preferred_element_type=jnp.float32).astype(o_ref.dtype)


def _matmul(a, w, out_dtype, n_block):
    m, k = a.shape
    _, n = w.shape
    grid = n // n_block
    return pl.pallas_call(
        _mm_body,
        grid=(grid,),
        out_shape=jax.ShapeDtypeStruct((m, n), out_dtype),
        in_specs=[
            pl.BlockSpec((m, k), lambda j: (0, 0)),
            pl.BlockSpec((k, n_block), lambda j: (0, j)),
        ],
        out_specs=pl.BlockSpec((m, n_block), lambda j: (0, j)),
        compiler_params=pltpu.CompilerParams(vmem_limit_bytes=VMEM_LIMIT),
    )(a, w)


HG = 8
BH = MH // S


def _attn_body(q_ref, k_ref, v_ref, qr_ref, kr_ref, o_ref):
    kr = kr_ref[...]
    scale = jnp.bfloat16(SCALE)
    ones_m = jnp.ones((S, 128), jnp.bfloat16)
    for h in range(HG):
        q = q_ref[:, h * Dh:(h + 1) * Dh] * scale
        k = k_ref[:, h * Dh:(h + 1) * Dh]
        qr = qr_ref[:, h * Dr:(h + 1) * Dr] * scale
        s = lax.dot_general(q, k, (((1,), (1,)), ((), ())),
                            preferred_element_type=jnp.float32)
        s = s + lax.dot_general(qr, kr, (((1,), (1,)), ((), ())),
                                preferred_element_type=jnp.float32)
        p = jnp.exp(s.astype(jnp.bfloat16))
        o = jnp.dot(p, v_ref[:, h * Dh:(h + 1) * Dh],
                    preferred_element_type=jnp.float32)
        ssum = jnp.dot(p, ones_m, preferred_element_type=jnp.float32)
        o_ref[:, h * Dh:(h + 1) * Dh] = (
            o * (1.0 / ssum[:, 0:1])).astype(jnp.bfloat16)


def _attention(q2d, k2d, v2d, qr2d, kr2d):
    return pl.pallas_call(
        _attn_body,
        grid=(BH, H // HG),
        out_shape=jax.ShapeDtypeStruct((MH, H * Dh), jnp.bfloat16),
        in_specs=[
            pl.BlockSpec((S, HG * Dh), lambda b, g: (b, g)),
            pl.BlockSpec((S, HG * Dh), lambda b, g: (b, g)),
            pl.BlockSpec((S, HG * Dh), lambda b, g: (b, g)),
            pl.BlockSpec((S, HG * Dr), lambda b, g: (b, g)),
            pl.BlockSpec((S, Dr), lambda b, g: (b, 0)),
        ],
        out_specs=pl.BlockSpec((S, HG * Dh), lambda b, g: (b, g)),
        compiler_params=pltpu.CompilerParams(vmem_limit_bytes=VMEM_LIMIT),
    )(q2d, k2d, v2d, qr2d, kr2d)


NC = 512
NCHUNK = D // NC


def _out_body(o_ref, wo_ref, out_ref,
              wo_buf, sbuf, rbufx, rbufy,
              dma_sems, sx_send, sx_recv, sy_send, sy_recv):
    my_x = lax.axis_index("x")
    my_y = lax.axis_index("y")
    xn = (1 - my_x, my_y)
    yn = (my_x, 1 - my_y)

    barrier_sem = pltpu.get_barrier_semaphore()
    for nbr in (xn, yn):
        pl.semaphore_signal(barrier_sem, inc=1, device_id=nbr,
                            device_id_type=pl.DeviceIdType.MESH)
    pl.semaphore_wait(barrier_sem, 2)

    def wo_copy(c, slot):
        return pltpu.make_async_copy(
            wo_ref.at[:, pl.ds(c * NC, NC)], wo_buf.at[slot],
            dma_sems.at[slot])

    def x_rdma(s):
        return pltpu.make_async_remote_copy(
            src_ref=sbuf.at[s], dst_ref=rbufx.at[s],
            send_sem=sx_send.at[s], recv_sem=sx_recv.at[s],
            device_id=xn, device_id_type=pl.DeviceIdType.MESH)

    def y_rdma(s):
        return pltpu.make_async_remote_copy(
            src_ref=rbufx.at[s], dst_ref=rbufy.at[s],
            send_sem=sy_send.at[s], recv_sem=sy_recv.at[s],
            device_id=yn, device_id_type=pl.DeviceIdType.MESH)

    def store(xv, yv, b0, c, val):
        pl.when(jnp.logical_and(my_x == xv, my_y == yv))(
            lambda: out_ref.__setitem__(
                (slice(b0, b0 + 2), slice(None),
                 slice(c * NC, (c + 1) * NC)),
                val.reshape(2, S, NC)))

    def handle_direct(s):
        x_rdma(s).wait_recv()
        y_rdma(s).start()
        val = rbufx[s].astype(jnp.float32)
        for xv in (0, 1):
            for yv in (0, 1):
                store(xv, yv, 2 * (1 - xv), 4 * yv + s, val)

    ORDER = [0, 4, 1, 5, 2, 6, 3, 7]
    wo_copy(ORDER[0], 0).start()
    for i, c in enumerate(ORDER):
        if i + 1 < NCHUNK:
            wo_copy(ORDER[i + 1], (i + 1) % 2).start()
        wo_copy(c, i % 2).wait()
        oj = jnp.dot(o_ref[...], wo_buf[i % 2].astype(jnp.bfloat16),
                     preferred_element_type=jnp.float32)
        for xv in (0, 1):
            for yv in (0, 1):
                store(xv, yv, 2 * xv, c, oj)
        s = c % 4
        cond = (my_y == 0) if c < 4 else (my_y == 1)

        def _send(s=s, oj=oj):
            sbuf[s] = oj.astype(jnp.bfloat16)
            x_rdma(s).start()

        pl.when(cond)(_send)
        if i in (3, 5, 7):
            handle_direct((i - 3) // 2)

    handle_direct(3)
    for s in range(4):
        y_rdma(s).wait_recv()
        val = rbufy[s].astype(jnp.float32)
        for xv in (0, 1):
            for yv in (0, 1):
                store(xv, yv, 2 * (1 - xv), 4 * (1 - yv) + s, val)
    for s in range(4):
        x_rdma(s).wait_send()
        y_rdma(s).wait_send()


def _out_proj_gather(o2d, Wo):
    return pl.pallas_call(
        _out_body,
        out_shape=jax.ShapeDtypeStruct((B, S, D), jnp.float32),
        in_specs=[
            pl.BlockSpec(memory_space=pltpu.VMEM),
            pl.BlockSpec(memory_space=pltpu.MemorySpace.HBM),
        ],
        out_specs=pl.BlockSpec(memory_space=pltpu.VMEM),
        scratch_shapes=[
            pltpu.VMEM((2, D, NC), jnp.float32),
            pltpu.VMEM((4, MH, NC), jnp.bfloat16),
            pltpu.VMEM((4, MH, NC), jnp.bfloat16),
            pltpu.VMEM((4, MH, NC), jnp.bfloat16),
            pltpu.SemaphoreType.DMA((2,)),
            pltpu.SemaphoreType.DMA((4,)),
            pltpu.SemaphoreType.DMA((4,)),
            pltpu.SemaphoreType.DMA((4,)),
            pltpu.SemaphoreType.DMA((4,)),
        ],
        compiler_params=pltpu.CompilerParams(
            collective_id=1, vmem_limit_bytes=VMEM_LIMIT),
    )(o2d, Wo)


def kernel(x, Wdkv, Wuk, Wuv, Wq, Wqr, Wkr, Wo):
    wdkv = Wdkv.astype(jnp.bfloat16)
    wuk = Wuk.astype(jnp.bfloat16)
    wuv = Wuv.astype(jnp.bfloat16)
    wkr = Wkr.astype(jnp.bfloat16)

    k2d, v2d, kr2d, xb, qr2d = _kv_exchange(x, wdkv, wuk, wuv, wkr, Wqr)
    q2d = _matmul(xb, Wq, jnp.bfloat16, 1024)

    o2d = _attention(q2d, k2d, v2d, qr2d, kr2d)
    return _out_proj_gather(o2d, Wo)


# device time: 156175 ns/iter; 1.6150x vs baseline; 1.0528x over previous
import jax
import jax.numpy as jnp
from jax import lax
from jax.experimental import pallas as pl
from jax.experimental.pallas import tpu as pltpu

B, S, H, Dh, Dr = 4, 256, 32, 128, 64
D = 4096
DC_SH = 128
M = B * S
MH = M // 2
SCALE = (Dh + Dr) ** -0.5
VMEM_LIMIT = 60 * 1024 * 1024


def _kv_body(x_ref, wdkv_ref, wuk_ref, wuv_ref, wkr_ref, wqr_ref,
             k_ref, v_ref, kr_ref, xb_ref, qr_ref,
             c_buf, c_full, wuk_full, wuv_full, wqr_buf,
             send_sems, recv_sems, qdma_sems):
    my_x = lax.axis_index("x")
    my_y = lax.axis_index("y")
    nbr = (my_x, 1 - my_y)

    barrier_sem = pltpu.get_barrier_semaphore()
    pl.semaphore_signal(barrier_sem, inc=1, device_id=nbr,
                        device_id_type=pl.DeviceIdType.MESH)
    pl.semaphore_wait(barrier_sem, 1)

    pl.when(my_x == 0)(
        lambda: xb_ref.__setitem__(
            ..., x_ref[0:2].reshape(MH, D).astype(jnp.bfloat16)))
    pl.when(my_x == 1)(
        lambda: xb_ref.__setitem__(
            ..., x_ref[2:4].reshape(MH, D).astype(jnp.bfloat16)))

    cm = jnp.dot(xb_ref[...], wdkv_ref[...],
                 preferred_element_type=jnp.float32).astype(jnp.bfloat16)

    def _start(slot):
        c_buf[slot] = cm
        row = slice(slot * DC_SH, (slot + 1) * DC_SH)
        wuk_full[row, :] = wuk_ref[...]
        wuv_full[row, :] = wuv_ref[...]
        pltpu.make_async_remote_copy(
            src_ref=c_buf.at[slot], dst_ref=c_buf.at[slot],
            send_sem=send_sems.at[2], recv_sem=recv_sems.at[2],
            device_id=nbr, device_id_type=pl.DeviceIdType.MESH).start()
        pltpu.make_async_remote_copy(
            src_ref=wuk_full.at[pl.ds(slot * DC_SH, DC_SH)],
            dst_ref=wuk_full.at[pl.ds(slot * DC_SH, DC_SH)],
            send_sem=send_sems.at[0], recv_sem=recv_sems.at[0],
            device_id=nbr, device_id_type=pl.DeviceIdType.MESH).start()
        pltpu.make_async_remote_copy(
            src_ref=wuv_full.at[pl.ds(slot * DC_SH, DC_SH)],
            dst_ref=wuv_full.at[pl.ds(slot * DC_SH, DC_SH)],
            send_sem=send_sems.at[1], recv_sem=recv_sems.at[1],
            device_id=nbr, device_id_type=pl.DeviceIdType.MESH).start()

    pl.when(my_y == 0)(lambda: _start(0))
    pl.when(my_y == 1)(lambda: _start(1))

    rdma_c = pltpu.make_async_remote_copy(
        src_ref=c_buf.at[0], dst_ref=c_buf.at[0],
        send_sem=send_sems.at[2], recv_sem=recv_sems.at[2],
        device_id=nbr, device_id_type=pl.DeviceIdType.MESH)
    rdma_wuk = pltpu.make_async_remote_copy(
        src_ref=wuk_full.at[pl.ds(0, DC_SH)],
        dst_ref=wuk_full.at[pl.ds(0, DC_SH)],
        send_sem=send_sems.at[0], recv_sem=recv_sems.at[0],
        device_id=nbr, device_id_type=pl.DeviceIdType.MESH)
    rdma_wuv = pltpu.make_async_remote_copy(
        src_ref=wuv_full.at[pl.ds(0, DC_SH)],
        dst_ref=wuv_full.at[pl.ds(0, DC_SH)],
        send_sem=send_sems.at[1], recv_sem=recv_sems.at[1],
        device_id=nbr, device_id_type=pl.DeviceIdType.MESH)

    kr_ref[...] = jnp.dot(
        xb_ref[...], wkr_ref[...],
        preferred_element_type=jnp.float32).astype(jnp.bfloat16)

    QC = 256
    nq = (H * Dr) // QC

    def qr_copy(j):
        return pltpu.make_async_copy(
            wqr_ref.at[:, pl.ds(j * QC, QC)], wqr_buf.at[j % 2],
            qdma_sems.at[j % 2])

    qr_copy(0).start()
    for j in range(nq):
        if j + 1 < nq:
            qr_copy(j + 1).start()
        qr_copy(j).wait()
        qr_ref[:, j * QC:(j + 1) * QC] = jnp.dot(
            xb_ref[...], wqr_buf[j % 2].astype(jnp.bfloat16),
            preferred_element_type=jnp.float32).astype(jnp.bfloat16)

    rdma_c.wait()
    rdma_wuk.wait()
    c_full[:, 0:DC_SH] = c_buf[0]
    c_full[:, DC_SH:2 * DC_SH] = c_buf[1]
    NB = 1024
    for j in range(0, H * Dh, NB):
        sl = pl.ds(j, NB)
        k_ref[:, sl] = jnp.dot(
            c_full[...], wuk_full[:, sl],
            preferred_element_type=jnp.float32).astype(jnp.bfloat16)
    rdma_wuv.wait()
    for j in range(0, H * Dh, NB):
        sl = pl.ds(j, NB)
        v_ref[:, sl] = jnp.dot(
            c_full[...], wuv_full[:, sl],
            preferred_element_type=jnp.float32).astype(jnp.bfloat16)


def _kv_exchange(x2d, wdkv, wuk, wuv, wkr, wqr):
    return pl.pallas_call(
        _kv_body,
        out_shape=[
            jax.ShapeDtypeStruct((MH, H * Dh), jnp.bfloat16),
            jax.ShapeDtypeStruct((MH, H * Dh), jnp.bfloat16),
            jax.ShapeDtypeStruct((MH, Dr), jnp.bfloat16),
            jax.ShapeDtypeStruct((MH, D), jnp.bfloat16),
            jax.ShapeDtypeStruct((MH, H * Dr), jnp.bfloat16),
        ],
        in_specs=[pl.BlockSpec(memory_space=pltpu.VMEM)] * 5
        + [pl.BlockSpec(memory_space=pltpu.MemorySpace.HBM)],
        out_specs=[pl.BlockSpec(memory_space=pltpu.VMEM)] * 5,
        scratch_shapes=[
            pltpu.VMEM((2, MH, DC_SH), jnp.bfloat16),
            pltpu.VMEM((MH, 2 * DC_SH), jnp.bfloat16),
            pltpu.VMEM((2 * DC_SH, H * Dh), jnp.bfloat16),
            pltpu.VMEM((2 * DC_SH, H * Dh), jnp.bfloat16),
            pltpu.VMEM((2, D, 256), jnp.float32),
            pltpu.SemaphoreType.DMA((3,)),
            pltpu.SemaphoreType.DMA((3,)),
            pltpu.SemaphoreType.DMA((2,)),
        ],
        compiler_params=pltpu.CompilerParams(
            collective_id=0, vmem_limit_bytes=VMEM_LIMIT),
    )(x2d, wdkv, wuk, wuv, wkr, wqr)


def _mm_body(a_ref, w_ref, o_ref):
    a = a_ref[...]
    half = o_ref.shape[1] // 2
    w1 = w_ref[:, 0:half].astype(jnp.bfloat16)
    w2 = w_ref[:, half:].astype(jnp.bfloat16)
    o_ref[:, 0:half] = jnp.dot(
        a, w1, preferred_element_type=jnp.float32).astype(o_ref.dtype)
    o_ref[:, half:] = jnp.dot(
        a, w2, preferred_element_type=jnp.float32).astype(o_ref.dtype)


def _matmul(a, w, out_dtype, n_block):
    m, k = a.shape
    _, n = w.shape
    grid = n // n_block
    return pl.pallas_call(
        _mm_body,
        grid=(grid,),
        out_shape=jax.ShapeDtypeStruct((m, n), out_dtype),
        in_specs=[
            pl.BlockSpec((m, k), lambda j: (0, 0)),
            pl.BlockSpec((k, n_block), lambda j: (0, j)),
        ],
        out_specs=pl.BlockSpec((m, n_block), lambda j: (0, j)),
        compiler_params=pltpu.CompilerParams(vmem_limit_bytes=VMEM_LIMIT),
    )(a, w)


HG = 8
BH = MH // S


def _attn_body(q_ref, k_ref, v_ref, qr_ref, kr_ref, o_ref):
    kr = kr_ref[...]
    scale = jnp.bfloat16(SCALE)
    ones_m = jnp.ones((S, 128), jnp.bfloat16)
    for h in range(HG):
        q = q_ref[:, h * Dh:(h + 1) * Dh] * scale
        k = k_ref[:, h * Dh:(h + 1) * Dh]
        qr = qr_ref[:, h * Dr:(h + 1) * Dr] * scale
        s = lax.dot_general(q, k, (((1,), (1,)), ((), ())),
                            preferred_element_type=jnp.float32)
        s = s + lax.dot_general(qr, kr, (((1,), (1,)), ((), ())),
                                preferred_element_type=jnp.float32)
        p = jnp.exp(s.astype(jnp.bfloat16))
        o = jnp.dot(p, v_ref[:, h * Dh:(h + 1) * Dh],
                    preferred_element_type=jnp.float32)
        ssum = jnp.dot(p, ones_m, preferred_element_type=jnp.float32)
        o_ref[:, h * Dh:(h + 1) * Dh] = (
            o * (1.0 / ssum[:, 0:1])).astype(jnp.bfloat16)


def _attention(q2d, k2d, v2d, qr2d, kr2d):
    return pl.pallas_call(
        _attn_body,
        grid=(BH, H // HG),
        out_shape=jax.ShapeDtypeStruct((MH, H * Dh), jnp.bfloat16),
        in_specs=[
            pl.BlockSpec((S, HG * Dh), lambda b, g: (b, g)),
            pl.BlockSpec((S, HG * Dh), lambda b, g: (b, g)),
            pl.BlockSpec((S, HG * Dh), lambda b, g: (b, g)),
            pl.BlockSpec((S, HG * Dr), lambda b, g: (b, g)),
            pl.BlockSpec((S, Dr), lambda b, g: (b, 0)),
        ],
        out_specs=pl.BlockSpec((S, HG * Dh), lambda b, g: (b, g)),
        compiler_params=pltpu.CompilerParams(vmem_limit_bytes=VMEM_LIMIT),
    )(q2d, k2d, v2d, qr2d, kr2d)


NC = 512
NCHUNK = D // NC


def _out_body(o_ref, wo_ref, out_ref,
              wo_buf, sbuf, rbufx, rbufy,
              dma_sems, sx_send, sx_recv, sy_send, sy_recv):
    my_x = lax.axis_index("x")
    my_y = lax.axis_index("y")
    xn = (1 - my_x, my_y)
    yn = (my_x, 1 - my_y)

    barrier_sem = pltpu.get_barrier_semaphore()
    for nbr in (xn, yn):
        pl.semaphore_signal(barrier_sem, inc=1, device_id=nbr,
                            device_id_type=pl.DeviceIdType.MESH)
    pl.semaphore_wait(barrier_sem, 2)

    def wo_copy(c, slot):
        return pltpu.make_async_copy(
            wo_ref.at[:, pl.ds(c * NC, NC)], wo_buf.at[slot],
            dma_sems.at[slot])

    def x_rdma(s):
        return pltpu.make_async_remote_copy(
            src_ref=sbuf.at[s], dst_ref=rbufx.at[s],
            send_sem=sx_send.at[s], recv_sem=sx_recv.at[s],
            device_id=xn, device_id_type=pl.DeviceIdType.MESH)

    def y_rdma(s):
        return pltpu.make_async_remote_copy(
            src_ref=rbufx.at[s], dst_ref=rbufy.at[s],
            send_sem=sy_send.at[s], recv_sem=sy_recv.at[s],
            device_id=yn, device_id_type=pl.DeviceIdType.MESH)

    def store(xv, yv, b0, c, val):
        pl.when(jnp.logical_and(my_x == xv, my_y == yv))(
            lambda: out_ref.__setitem__(
                (slice(b0, b0 + 2), slice(None),
                 slice(c * NC, (c + 1) * NC)),
                val.reshape(2, S, NC)))

    def handle_direct(s):
        x_rdma(s).wait_recv()
        y_rdma(s).start()
        val = rbufx[s]
        for xv in (0, 1):
            for yv in (0, 1):
                store(xv, yv, 2 * (1 - xv), 4 * yv + s, val)

    ORDER = [0, 4, 1, 5, 2, 6, 3, 7]
    wo_copy(ORDER[0], 0).start()
    for i, c in enumerate(ORDER):
        if i + 1 < NCHUNK:
            wo_copy(ORDER[i + 1], (i + 1) % 2).start()
        wo_copy(c, i % 2).wait()
        oj = jnp.dot(o_ref[...], wo_buf[i % 2].astype(jnp.bfloat16),
                     preferred_element_type=jnp.float32).astype(jnp.bfloat16)
        for xv in (0, 1):
            for yv in (0, 1):
                store(xv, yv, 2 * xv, c, oj)
        s = c % 4
        cond = (my_y == 0) if c < 4 else (my_y == 1)

        def _send(s=s, oj=oj):
            sbuf[s] = oj
            x_rdma(s).start()

        pl.when(cond)(_send)
        if i in (3, 5, 7):
            handle_direct((i - 3) // 2)

    handle_direct(3)
    for s in range(4):
        y_rdma(s).wait_recv()
        val = rbufy[s]
        for xv in (0, 1):
            for yv in (0, 1):
                store(xv, yv, 2 * (1 - xv), 4 * (1 - yv) + s, val)
    for s in range(4):
        x_rdma(s).wait_send()
        y_rdma(s).wait_send()


def _out_proj_gather(o2d, Wo):
    return pl.pallas_call(
        _out_body,
        out_shape=jax.ShapeDtypeStruct((B, S, D), jnp.bfloat16),
        in_specs=[
            pl.BlockSpec(memory_space=pltpu.VMEM),
            pl.BlockSpec(memory_space=pltpu.MemorySpace.HBM),
        ],
        out_specs=pl.BlockSpec(memory_space=pltpu.VMEM),
        scratch_shapes=[
            pltpu.VMEM((2, D, NC), jnp.float32),
            pltpu.VMEM((4, MH, NC), jnp.bfloat16),
            pltpu.VMEM((4, MH, NC), jnp.bfloat16),
            pltpu.VMEM((4, MH, NC), jnp.bfloat16),
            pltpu.SemaphoreType.DMA((2,)),
            pltpu.SemaphoreType.DMA((4,)),
            pltpu.SemaphoreType.DMA((4,)),
            pltpu.SemaphoreType.DMA((4,)),
            pltpu.SemaphoreType.DMA((4,)),
        ],
        compiler_params=pltpu.CompilerParams(
            collective_id=1, vmem_limit_bytes=VMEM_LIMIT),
    )(o2d, Wo)


def kernel(x, Wdkv, Wuk, Wuv, Wq, Wqr, Wkr, Wo):
    wdkv = Wdkv.astype(jnp.bfloat16)
    wuk = Wuk.astype(jnp.bfloat16)
    wuv = Wuv.astype(jnp.bfloat16)
    wkr = Wkr.astype(jnp.bfloat16)

    k2d, v2d, kr2d, xb, qr2d = _kv_exchange(x, wdkv, wuk, wuv, wkr, Wqr)
    q2d = _matmul(xb, Wq, jnp.bfloat16, 1024)

    o2d = _attention(q2d, k2d, v2d, qr2d, kr2d)
    return _out_proj_gather(o2d, Wo)


# device time: 155581 ns/iter; 1.6211x vs baseline; 1.0038x over previous
import jax
import jax.numpy as jnp
from jax import lax
from jax.experimental import pallas as pl
from jax.experimental.pallas import tpu as pltpu

B, S, H, Dh, Dr = 4, 256, 32, 128, 64
D = 4096
DC_SH = 128
M = B * S
MH = M // 2
SCALE = (Dh + Dr) ** -0.5
VMEM_LIMIT = 60 * 1024 * 1024


def _kv_body(x_ref, wdkv_ref, wuk_ref, wuv_ref, wkr_ref, wqr_ref,
             k_ref, v_ref, kr_ref, xb_ref, qr_ref,
             c_buf, c_full, wuk_full, wuv_full, wqr_buf,
             send_sems, recv_sems, qdma_sems):
    my_x = lax.axis_index("x")
    my_y = lax.axis_index("y")
    nbr = (my_x, 1 - my_y)

    barrier_sem = pltpu.get_barrier_semaphore()
    pl.semaphore_signal(barrier_sem, inc=1, device_id=nbr,
                        device_id_type=pl.DeviceIdType.MESH)
    pl.semaphore_wait(barrier_sem, 1)

    def _start_wuk(slot):
        row = slice(slot * DC_SH, (slot + 1) * DC_SH)
        wuk_full[row, :] = wuk_ref[...]
        pltpu.make_async_remote_copy(
            src_ref=wuk_full.at[pl.ds(slot * DC_SH, DC_SH)],
            dst_ref=wuk_full.at[pl.ds(slot * DC_SH, DC_SH)],
            send_sem=send_sems.at[0], recv_sem=recv_sems.at[0],
            device_id=nbr, device_id_type=pl.DeviceIdType.MESH).start()

    pl.when(my_y == 0)(lambda: _start_wuk(0))
    pl.when(my_y == 1)(lambda: _start_wuk(1))

    pl.when(my_x == 0)(
        lambda: xb_ref.__setitem__(
            ..., x_ref[0:2].reshape(MH, D).astype(jnp.bfloat16)))
    pl.when(my_x == 1)(
        lambda: xb_ref.__setitem__(
            ..., x_ref[2:4].reshape(MH, D).astype(jnp.bfloat16)))

    cm = jnp.dot(xb_ref[...], wdkv_ref[...],
                 preferred_element_type=jnp.float32).astype(jnp.bfloat16)

    def _start_c_wuv(slot):
        c_buf[slot] = cm
        pltpu.make_async_remote_copy(
            src_ref=c_buf.at[slot], dst_ref=c_buf.at[slot],
            send_sem=send_sems.at[2], recv_sem=recv_sems.at[2],
            device_id=nbr, device_id_type=pl.DeviceIdType.MESH).start()
        row = slice(slot * DC_SH, (slot + 1) * DC_SH)
        wuv_full[row, :] = wuv_ref[...]
        pltpu.make_async_remote_copy(
            src_ref=wuv_full.at[pl.ds(slot * DC_SH, DC_SH)],
            dst_ref=wuv_full.at[pl.ds(slot * DC_SH, DC_SH)],
            send_sem=send_sems.at[1], recv_sem=recv_sems.at[1],
            device_id=nbr, device_id_type=pl.DeviceIdType.MESH).start()

    pl.when(my_y == 0)(lambda: _start_c_wuv(0))
    pl.when(my_y == 1)(lambda: _start_c_wuv(1))

    rdma_c = pltpu.make_async_remote_copy(
        src_ref=c_buf.at[0], dst_ref=c_buf.at[0],
        send_sem=send_sems.at[2], recv_sem=recv_sems.at[2],
        device_id=nbr, device_id_type=pl.DeviceIdType.MESH)
    rdma_wuk = pltpu.make_async_remote_copy(
        src_ref=wuk_full.at[pl.ds(0, DC_SH)],
        dst_ref=wuk_full.at[pl.ds(0, DC_SH)],
        send_sem=send_sems.at[0], recv_sem=recv_sems.at[0],
        device_id=nbr, device_id_type=pl.DeviceIdType.MESH)
    rdma_wuv = pltpu.make_async_remote_copy(
        src_ref=wuv_full.at[pl.ds(0, DC_SH)],
        dst_ref=wuv_full.at[pl.ds(0, DC_SH)],
        send_sem=send_sems.at[1], recv_sem=recv_sems.at[1],
        device_id=nbr, device_id_type=pl.DeviceIdType.MESH)

    kr_ref[...] = jnp.dot(
        xb_ref[...], wkr_ref[...],
        preferred_element_type=jnp.float32).astype(jnp.bfloat16)

    QC = 256
    nq = (H * Dr) // QC

    def qr_copy(j):
        return pltpu.make_async_copy(
            wqr_ref.at[:, pl.ds(j * QC, QC)], wqr_buf.at[j % 2],
            qdma_sems.at[j % 2])

    qr_copy(0).start()
    for j in range(nq):
        if j + 1 < nq:
            qr_copy(j + 1).start()
        qr_copy(j).wait()
        qr_ref[:, j * QC:(j + 1) * QC] = jnp.dot(
            xb_ref[...], wqr_buf[j % 2].astype(jnp.bfloat16),
            preferred_element_type=jnp.float32).astype(jnp.bfloat16)

    rdma_c.wait()
    rdma_wuk.wait()
    c_full[:, 0:DC_SH] = c_buf[0]
    c_full[:, DC_SH:2 * DC_SH] = c_buf[1]
    NB = 1024
    for j in range(0, H * Dh, NB):
        sl = pl.ds(j, NB)
        k_ref[:, sl] = jnp.dot(
            c_full[...], wuk_full[:, sl],
            preferred_element_type=jnp.float32).astype(jnp.bfloat16)
    rdma_wuv.wait()
    for j in range(0, H * Dh, NB):
        sl = pl.ds(j, NB)
        v_ref[:, sl] = jnp.dot(
            c_full[...], wuv_full[:, sl],
            preferred_element_type=jnp.float32).astype(jnp.bfloat16)


def _kv_exchange(x2d, wdkv, wuk, wuv, wkr, wqr):
    return pl.pallas_call(
        _kv_body,
        out_shape=[
            jax.ShapeDtypeStruct((MH, H * Dh), jnp.bfloat16),
            jax.ShapeDtypeStruct((MH, H * Dh), jnp.bfloat16),
            jax.ShapeDtypeStruct((MH, Dr), jnp.bfloat16),
            jax.ShapeDtypeStruct((MH, D), jnp.bfloat16),
            jax.ShapeDtypeStruct((MH, H * Dr), jnp.bfloat16),
        ],
        in_specs=[pl.BlockSpec(memory_space=pltpu.VMEM)] * 5
        + [pl.BlockSpec(memory_space=pltpu.MemorySpace.HBM)],
        out_specs=[pl.BlockSpec(memory_space=pltpu.VMEM)] * 5,
        scratch_shapes=[
            pltpu.VMEM((2, MH, DC_SH), jnp.bfloat16),
            pltpu.VMEM((MH, 2 * DC_SH), jnp.bfloat16),
            pltpu.VMEM((2 * DC_SH, H * Dh), jnp.bfloat16),
            pltpu.VMEM((2 * DC_SH, H * Dh), jnp.bfloat16),
            pltpu.VMEM((2, D, 256), jnp.float32),
            pltpu.SemaphoreType.DMA((3,)),
            pltpu.SemaphoreType.DMA((3,)),
            pltpu.SemaphoreType.DMA((2,)),
        ],
        compiler_params=pltpu.CompilerParams(
            collective_id=0, vmem_limit_bytes=VMEM_LIMIT),
    )(x2d, wdkv, wuk, wuv, wkr, wqr)


def _mm_body(a_ref, w_ref, o_ref):
    a = a_ref[...]
    half = o_ref.shape[1] // 2
    w1 = w_ref[:, 0:half].astype(jnp.bfloat16)
    w2 = w_ref[:, half:].astype(jnp.bfloat16)
    o_ref[:, 0:half] = jnp.dot(
        a, w1, preferred_element_type=jnp.float32).astype(o_ref.dtype)
    o_ref[:, half:] = jnp.dot(
        a, w2, preferred_element_type=jnp.float32).astype(o_ref.dtype)


def _matmul(a, w, out_dtype, n_block):
    m, k = a.shape
    _, n = w.shape
    grid = n // n_block
    return pl.pallas_call(
        _mm_body,
        grid=(grid,),
        out_shape=jax.ShapeDtypeStruct((m, n), out_dtype),
        in_specs=[
            pl.BlockSpec((m, k), lambda j: (0, 0)),
            pl.BlockSpec((k, n_block), lambda j: (0, j)),
        ],
        out_specs=pl.BlockSpec((m, n_block), lambda j: (0, j)),
        compiler_params=pltpu.CompilerParams(vmem_limit_bytes=VMEM_LIMIT),
    )(a, w)


HG = 8
BH = MH // S


def _attn_body(q_ref, k_ref, v_ref, qr_ref, kr_ref, o_ref):
    kr = kr_ref[...]
    scale = jnp.bfloat16(SCALE)
    ones_m = jnp.ones((S, 128), jnp.bfloat16)
    for h in range(HG):
        q = q_ref[:, h * Dh:(h + 1) * Dh] * scale
        k = k_ref[:, h * Dh:(h + 1) * Dh]
        qr = qr_ref[:, h * Dr:(h + 1) * Dr] * scale
        s = lax.dot_general(q, k, (((1,), (1,)), ((), ())),
                            preferred_element_type=jnp.float32)
        s = s + lax.dot_general(qr, kr, (((1,), (1,)), ((), ())),
                                preferred_element_type=jnp.float32)
        p = jnp.exp(s.astype(jnp.bfloat16))
        o = jnp.dot(p, v_ref[:, h * Dh:(h + 1) * Dh],
                    preferred_element_type=jnp.float32)
        ssum = jnp.dot(p, ones_m, preferred_element_type=jnp.float32)
        o_ref[:, h * Dh:(h + 1) * Dh] = (
            o * (1.0 / ssum[:, 0:1])).astype(jnp.bfloat16)


def _attention(q2d, k2d, v2d, qr2d, kr2d):
    return pl.pallas_call(
        _attn_body,
        grid=(BH, H // HG),
        out_shape=jax.ShapeDtypeStruct((MH, H * Dh), jnp.bfloat16),
        in_specs=[
            pl.BlockSpec((S, HG * Dh), lambda b, g: (b, g)),
            pl.BlockSpec((S, HG * Dh), lambda b, g: (b, g)),
            pl.BlockSpec((S, HG * Dh), lambda b, g: (b, g)),
            pl.BlockSpec((S, HG * Dr), lambda b, g: (b, g)),
            pl.BlockSpec((S, Dr), lambda b, g: (b, 0)),
        ],
        out_specs=pl.BlockSpec((S, HG * Dh), lambda b, g: (b, g)),
        compiler_params=pltpu.CompilerParams(vmem_limit_bytes=VMEM_LIMIT),
    )(q2d, k2d, v2d, qr2d, kr2d)


NC = 512
NCHUNK = D // NC


def _out_body(o_ref, wo_ref, out_ref,
              wo_buf, sbuf, rbufx, rbufy,
              dma_sems, sx_send, sx_recv, sy_send, sy_recv):
    my_x = lax.axis_index("x")
    my_y = lax.axis_index("y")
    xn = (1 - my_x, my_y)
    yn = (my_x, 1 - my_y)

    barrier_sem = pltpu.get_barrier_semaphore()
    for nbr in (xn, yn):
        pl.semaphore_signal(barrier_sem, inc=1, device_id=nbr,
                            device_id_type=pl.DeviceIdType.MESH)
    pl.semaphore_wait(barrier_sem, 2)

    def wo_copy(c, slot):
        return pltpu.make_async_copy(
            wo_ref.at[:, pl.ds(c * NC, NC)], wo_buf.at[slot],
            dma_sems.at[slot])

    def x_rdma(s):
        return pltpu.make_async_remote_copy(
            src_ref=sbuf.at[s], dst_ref=rbufx.at[s],
            send_sem=sx_send.at[s], recv_sem=sx_recv.at[s],
            device_id=xn, device_id_type=pl.DeviceIdType.MESH)

    def y_rdma(s):
        return pltpu.make_async_remote_copy(
            src_ref=rbufx.at[s], dst_ref=rbufy.at[s],
            send_sem=sy_send.at[s], recv_sem=sy_recv.at[s],
            device_id=yn, device_id_type=pl.DeviceIdType.MESH)

    def store(xv, yv, b0, c, val):
        pl.when(jnp.logical_and(my_x == xv, my_y == yv))(
            lambda: out_ref.__setitem__(
                (slice(b0, b0 + 2), slice(None),
                 slice(c * NC, (c + 1) * NC)),
                val.reshape(2, S, NC)))

    def handle_direct(s):
        x_rdma(s).wait_recv()
        y_rdma(s).start()
        val = rbufx[s]
        for xv in (0, 1):
            for yv in (0, 1):
                store(xv, yv, 2 * (1 - xv), 4 * yv + s, val)

    ORDER = [0, 4, 1, 5, 2, 6, 3, 7]
    wo_copy(ORDER[0], 0).start()
    for i, c in enumerate(ORDER):
        if i + 1 < NCHUNK:
            wo_copy(ORDER[i + 1], (i + 1) % 2).start()
        wo_copy(c, i % 2).wait()
        oj = jnp.dot(o_ref[...], wo_buf[i % 2].astype(jnp.bfloat16),
                     preferred_element_type=jnp.float32).astype(jnp.bfloat16)
        for xv in (0, 1):
            for yv in (0, 1):
                store(xv, yv, 2 * xv, c, oj)
        s = c % 4
        cond = (my_y == 0) if c < 4 else (my_y == 1)

        def _send(s=s, oj=oj):
            sbuf[s] = oj
            x_rdma(s).start()

        pl.when(cond)(_send)
        if i in (3, 5, 7):
            handle_direct((i - 3) // 2)

    handle_direct(3)
    for s in range(4):
        y_rdma(s).wait_recv()
        val = rbufy[s]
        for xv in (0, 1):
            for yv in (0, 1):
                store(xv, yv, 2 * (1 - xv), 4 * (1 - yv) + s, val)
    for s in range(4):
        x_rdma(s).wait_send()
        y_rdma(s).wait_send()


def _out_proj_gather(o2d, Wo):
    return pl.pallas_call(
        _out_body,
        out_shape=jax.ShapeDtypeStruct((B, S, D), jnp.bfloat16),
        in_specs=[
            pl.BlockSpec(memory_space=pltpu.VMEM),
            pl.BlockSpec(memory_space=pltpu.MemorySpace.HBM),
        ],
        out_specs=pl.BlockSpec(memory_space=pltpu.VMEM),
        scratch_shapes=[
            pltpu.VMEM((2, D, NC), jnp.float32),
            pltpu.VMEM((4, MH, NC), jnp.bfloat16),
            pltpu.VMEM((4, MH, NC), jnp.bfloat16),
            pltpu.VMEM((4, MH, NC), jnp.bfloat16),
            pltpu.SemaphoreType.DMA((2,)),
            pltpu.SemaphoreType.DMA((4,)),
            pltpu.SemaphoreType.DMA((4,)),
            pltpu.SemaphoreType.DMA((4,)),
            pltpu.SemaphoreType.DMA((4,)),
        ],
        compiler_params=pltpu.CompilerParams(
            collective_id=1, vmem_limit_bytes=VMEM_LIMIT),
    )(o2d, Wo)


def kernel(x, Wdkv, Wuk, Wuv, Wq, Wqr, Wkr, Wo):
    wdkv = Wdkv.astype(jnp.bfloat16)
    wuk = Wuk.astype(jnp.bfloat16)
    wuv = Wuv.astype(jnp.bfloat16)
    wkr = Wkr.astype(jnp.bfloat16)

    k2d, v2d, kr2d, xb, qr2d = _kv_exchange(x, wdkv, wuk, wuv, wkr, Wqr)
    q2d = _matmul(xb, Wq, jnp.bfloat16, 1024)

    o2d = _attention(q2d, k2d, v2d, qr2d, kr2d)
    return _out_proj_gather(o2d, Wo)
